# baseline (speedup 1.0000x reference)
import jax
import jax.numpy as jnp

N_USER = 50000
N_ITEM = 50000
D = 64


def _bn(x, g, b):
    m = x.mean(axis=0)
    v = x.var(axis=0)
    return (x - m) / jnp.sqrt(v + 1e-5) * g + b


def _evolve(adj, t_diff, xu, xi, eu, ei, Wu, Wi):
    s, d = adj[0], adj[1]
    msg_u = jax.ops.segment_sum(jnp.take(xi, d, axis=0), s, num_segments=N_USER)
    msg_i = jax.ops.segment_sum(jnp.take(xu, s, axis=0), d, num_segments=N_ITEM)
    decay = jnp.exp(-t_diff)
    xu2 = jnp.tanh((decay * xu + msg_u) @ Wu) + eu
    xi2 = jnp.tanh((decay * xi + msg_i) @ Wi) + ei
    return xu2, xi2


def _pmtl(a, b, Wz, Wh, Wc):
    h = jnp.concatenate([a, b], axis=1)
    return jnp.tanh(h @ Wz), jnp.tanh(h @ Wh), jnp.tanh(h @ Wc)


def _predict(zu, zi, Wpu, Wpi):
    pu = zu @ Wpu
    pi = zi @ Wpi
    if pu.ndim == 2 and pi.ndim == 2:
        return jnp.sum(pu * pi, axis=-1, keepdims=True)
    if pi.ndim == 3:
        return jnp.einsum('bd,bkd->bk', pu, pi)
    return jnp.einsum('bkd,bd->bk', pu, pi)


def _update(xu, xi, adj_i2u, adj_u2i, Wu, Wi):
    mu = jax.ops.segment_sum(jnp.take(xi, adj_i2u[0], axis=0), adj_i2u[1], num_segments=N_USER)
    mi = jax.ops.segment_sum(jnp.take(xu, adj_u2i[0], axis=0), adj_u2i[1], num_segments=N_ITEM)
    return jnp.tanh(mu @ Wu), jnp.tanh(mi @ Wi)


def kernel(t_diff, adj_his, adj_ctx, adj_tgt_i2u, adj_tgt_u2i, tgt_u, tgt_i, tgt_u_neg, tgt_i_neg, xu_in_his, xi_in_his, xu_in_ctx, xi_in_ctx, embeds_u, embeds_i, g_uh, b_uh, g_ih, b_ih, g_uc, b_uc, g_ic, b_ic, Wu_eh, Wi_eh, Wu_ec, Wi_ec, Wu_uh, Wi_uh, Wu_uc, Wi_uc, Wz_u, Wh_u, Wc_u, Wz_i, Wh_i, Wc_i, Wp_u, Wp_i):
    xu_in_his = _bn(xu_in_his, g_uh, b_uh)
    xi_in_his = _bn(xi_in_his, g_ih, b_ih)
    xu_in_ctx = _bn(xu_in_ctx, g_uc, b_uc)
    xi_in_ctx = _bn(xi_in_ctx, g_ic, b_ic)
    xu_mh, xi_mh = _evolve(adj_his, t_diff, xu_in_his, xi_in_his, embeds_u, embeds_i, Wu_eh, Wi_eh)
    xu_mc, xi_mc = _evolve(adj_ctx, t_diff, xu_in_ctx, xi_in_ctx, embeds_u, embeds_i, Wu_ec, Wi_ec)
    zu_t, xu_th, xu_tc = _pmtl(xu_mh, xu_mc, Wz_u, Wh_u, Wc_u)
    zi_t, xi_th, xi_tc = _pmtl(xi_mh, xi_mc, Wz_i, Wh_i, Wc_i)
    zu_enc = jnp.concatenate([zu_t, embeds_u], axis=1)
    zi_enc = jnp.concatenate([zi_t, embeds_i], axis=1)
    zu_pos = jnp.take(zu_enc, tgt_u, axis=0)
    zu_neg = jnp.take(zu_enc, tgt_u_neg, axis=0)
    zi_pos = jnp.take(zi_enc, tgt_i, axis=0)
    zi_neg = jnp.take(zi_enc, tgt_i_neg, axis=0)
    pos = _predict(zu_pos, zi_pos, Wp_u, Wp_i)
    neg_u = _predict(zu_pos, zi_neg, Wp_u, Wp_i)
    neg_i = _predict(zu_neg, zi_pos, Wp_u, Wp_i)
    scores = jnp.concatenate([pos, neg_u, neg_i], axis=-1)
    loss = -jax.nn.log_softmax(scores, axis=1)[:, 0].mean()
    dxu_h, dxi_h = _update(xu_th, xi_th, adj_tgt_i2u, adj_tgt_u2i, Wu_uh, Wi_uh)
    dxu_c, dxi_c = _update(xu_tc, xi_tc, adj_tgt_i2u, adj_tgt_u2i, Wu_uc, Wi_uc)
    return (loss, zu_pos, zi_enc, xu_th + dxu_h, xi_th + dxi_h, xu_tc + dxu_c, xi_tc + dxi_c)


# speedup vs baseline: 5.1571x; 5.1571x over previous
import functools

import jax
import jax.numpy as jnp
from jax import lax
from jax.experimental import pallas as pl
from jax.experimental.pallas import tpu as pltpu
from jax.experimental.pallas import tpu_sc as plsc

NU = 50000
NI = 50000
D = 64
E = 800000
B = 4096
K = 5

NC = 2
NS = 16
HALF = NU // NC
ACC_ROWS = 25088
STRIPE = 1568
LAST_STRIPE = HALF - (NS - 1) * STRIPE
E_PAD = 802816
NCHUNK = 49

_f32 = jnp.float32
_i32 = jnp.int32


def _sc_edge_phase(srcr, dstr, tab, outr, zrows, src_v, dst_v, ldst, rows,
                   acc, gsem, ssem, cid, sid, n_chunks):
    base_row = cid * HALF
    trash = HALF + 8 + sid * 4

    pltpu.sync_copy(zrows, acc.at[pl.ds(sid * STRIPE, STRIPE)])
    plsc.subcore_barrier()

    def chunk(i, carry):
        erow = sid * (n_chunks * 8) + i * 8
        pltpu.sync_copy(srcr.at[pl.ds(erow, 8)], src_v)
        pltpu.sync_copy(dstr.at[pl.ds(erow, 8)], dst_v)
        gd = [None] * 8
        sd = [None] * 8
        gd[0] = pltpu.async_copy(tab.at[src_v.at[0]],
                                 rows.at[pl.ds(0, 128)], gsem)
        for j in range(8):
            for t in range(8):
                v = dst_v[j, pl.ds(t * 16, 16)]
                tl = v - base_row
                m = (tl >= 0) & (tl < HALF)
                ldst[j, pl.ds(t * 16, 16)] = jnp.where(m, tl, trash)
            if j + 1 < 8:
                if j >= 1:
                    sd[j - 1].wait()
                gd[j + 1] = pltpu.async_copy(
                    tab.at[src_v.at[j + 1]],
                    rows.at[pl.ds(((j + 1) % 2) * 128, 128)], gsem)
            gd[j].wait()
            sd[j] = pltpu.async_copy(rows.at[pl.ds((j % 2) * 128, 128)],
                                     acc.at[ldst.at[j]], ssem, add=True)
        sd[6].wait()
        sd[7].wait()
        return carry

    lax.fori_loop(0, n_chunks, chunk, 0)
    plsc.subcore_barrier()

    @pl.when(sid < NS - 1)
    def _():
        pltpu.sync_copy(acc.at[pl.ds(sid * STRIPE, STRIPE)],
                        outr.at[pl.ds(base_row + sid * STRIPE, STRIPE)])

    @pl.when(sid == NS - 1)
    def _():
        pltpu.sync_copy(acc.at[pl.ds((NS - 1) * STRIPE, LAST_STRIPE)],
                        outr.at[pl.ds(base_row + (NS - 1) * STRIPE,
                                      LAST_STRIPE)])


def _sc_msgs_body(s_h_s, s_h_d, d_h_s, d_h_d, s_c_s, s_c_d, d_c_s, d_c_d,
                  tb_ih, tb_uh, tb_ic, tb_uc, zrows,
                  o_uh, o_ih, o_uc, o_ic,
                  src_v, dst_v, ldst, rows, acc, gsem, ssem):
    cid = lax.axis_index("c")
    sid = lax.axis_index("s")
    phases = (
        (d_h_s, s_h_d, tb_ih, o_uh),
        (s_h_s, d_h_d, tb_uh, o_ih),
        (d_c_s, s_c_d, tb_ic, o_uc),
        (s_c_s, d_c_d, tb_uc, o_ic),
    )
    for srcr, dstr, tab, outr in phases:
        _sc_edge_phase(srcr, dstr, tab, outr, zrows, src_v, dst_v, ldst,
                       rows, acc, gsem, ssem, cid, sid, NCHUNK)


_sc_msgs = functools.partial(
    pl.kernel,
    out_type=tuple(jax.ShapeDtypeStruct((NU, D), _f32) for _ in range(4)),
    mesh=plsc.VectorSubcoreMesh(core_axis_name="c", subcore_axis_name="s"),
    scratch_types=[
        pltpu.VMEM((8, 128), _i32),
        pltpu.VMEM((8, 128), _i32),
        pltpu.VMEM((8, 128), _i32),
        pltpu.VMEM((256, D), _f32),
        pltpu.VMEM_SHARED((ACC_ROWS, D), _f32),
        pltpu.SemaphoreType.DMA,
        pltpu.SemaphoreType.DMA,
    ],
    compiler_params=pltpu.CompilerParams(use_tc_tiling_on_sc=False),
)(_sc_msgs_body)


def _sc_targets_body(zu_enc, zi_enc, xu_th, xi_th, xu_tc, xi_tc,
                     tgtu, tgti, tgtun, tgtin,
                     i2u_s, i2u_d, u2i_s, u2i_d, zrows,
                     zu_pos, zi_pos, zu_neg, zi_neg,
                     mu_h, mi_h, mu_c, mi_c,
                     idx_v, rows128, rows64, ldst, acc, gsem, ssem):
    cid = lax.axis_index("c")
    sid = lax.axis_index("s")
    wid = sid * NC + cid

    for idx2d, tab, outr, n_per_w in (
            (tgtu, zu_enc, zu_pos, 1),
            (tgti, zi_enc, zi_pos, 1),
            (tgtun, zu_enc, zu_neg, 5),
            (tgtin, zi_enc, zi_neg, 5),
    ):
        for q in range(n_per_w):
            crow = wid * n_per_w + q
            pltpu.sync_copy(idx2d.at[pl.ds(crow, 1)], idx_v.at[pl.ds(0, 1)])
            pltpu.async_copy(tab.at[idx_v.at[0]], rows128, gsem).wait()
            pltpu.sync_copy(rows128, outr.at[pl.ds(crow * 128, 128)])

    base_row = cid * HALF
    trash = HALF + 8 + sid * 4
    for src2d, dst2d, tab, outr in (
            (i2u_s, i2u_d, xi_th, mu_h),
            (u2i_s, u2i_d, xu_th, mi_h),
            (i2u_s, i2u_d, xi_tc, mu_c),
            (u2i_s, u2i_d, xu_tc, mi_c),
    ):
        pltpu.sync_copy(zrows, acc.at[pl.ds(sid * STRIPE, STRIPE)])
        plsc.subcore_barrier()
        for q in range(2):
            crow = sid * 2 + q
            pltpu.sync_copy(src2d.at[pl.ds(crow, 1)], idx_v.at[pl.ds(0, 1)])
            pltpu.sync_copy(dst2d.at[pl.ds(crow, 1)], idx_v.at[pl.ds(1, 1)])
            dsc = pltpu.async_copy(tab.at[idx_v.at[0]], rows64, gsem)
            for t in range(8):
                v = idx_v[1, pl.ds(t * 16, 16)]
                tl = v - base_row
                m = (tl >= 0) & (tl < HALF)
                ldst[0, pl.ds(t * 16, 16)] = jnp.where(m, tl, trash)
            dsc.wait()
            pltpu.async_copy(rows64, acc.at[ldst.at[0]], ssem,
                             add=True).wait()
        plsc.subcore_barrier()

        @pl.when(sid < NS - 1)
        def _():
            pltpu.sync_copy(acc.at[pl.ds(sid * STRIPE, STRIPE)],
                            outr.at[pl.ds(base_row + sid * STRIPE, STRIPE)])

        @pl.when(sid == NS - 1)
        def _():
            pltpu.sync_copy(acc.at[pl.ds((NS - 1) * STRIPE, LAST_STRIPE)],
                            outr.at[pl.ds(base_row + (NS - 1) * STRIPE,
                                          LAST_STRIPE)])


_sc_targets = functools.partial(
    pl.kernel,
    out_type=(
        jax.ShapeDtypeStruct((B, 2 * D), _f32),
        jax.ShapeDtypeStruct((B, 2 * D), _f32),
        jax.ShapeDtypeStruct((B * K, 2 * D), _f32),
        jax.ShapeDtypeStruct((B * K, 2 * D), _f32),
        jax.ShapeDtypeStruct((NU, D), _f32),
        jax.ShapeDtypeStruct((NI, D), _f32),
        jax.ShapeDtypeStruct((NU, D), _f32),
        jax.ShapeDtypeStruct((NI, D), _f32),
    ),
    mesh=plsc.VectorSubcoreMesh(core_axis_name="c", subcore_axis_name="s"),
    scratch_types=[
        pltpu.VMEM((8, 128), _i32),
        pltpu.VMEM((128, 2 * D), _f32),
        pltpu.VMEM((128, D), _f32),
        pltpu.VMEM((8, 128), _i32),
        pltpu.VMEM_SHARED((ACC_ROWS, D), _f32),
        pltpu.SemaphoreType.DMA,
        pltpu.SemaphoreType.DMA,
    ],
    compiler_params=pltpu.CompilerParams(use_tc_tiling_on_sc=False),
)(_sc_targets_body)


_RB = 1000
_NB = NU // _RB


def _stats_body(a, b, c, d, o):
    i = pl.program_id(0)

    @pl.when(i == 0)
    def _():
        o[...] = jnp.zeros_like(o)

    for r, x in enumerate((a, b, c, d)):
        xv = x[...]
        o[pl.ds(2 * r, 1), :] += jnp.sum(xv, 0, keepdims=True)
        o[pl.ds(2 * r + 1, 1), :] += jnp.sum(xv * xv, 0, keepdims=True)


def _tc_stats(xuh, xih, xuc, xic):
    spec = pl.BlockSpec((_RB, D), lambda i: (i, 0))
    return pl.pallas_call(
        _stats_body,
        grid=(_NB,),
        in_specs=[spec] * 4,
        out_specs=pl.BlockSpec((8, D), lambda i: (0, 0)),
        out_shape=jax.ShapeDtypeStruct((8, D), _f32),
    )(xuh, xih, xuc, xic)


def _apply_body(a, b, c, d, sc, sh, oa, ob, oc, od):
    for r, (x, o) in enumerate(((a, oa), (b, ob), (c, oc), (d, od))):
        o[...] = x[...] * sc[pl.ds(r, 1), :] + sh[pl.ds(r, 1), :]


def _tc_apply(xuh, xih, xuc, xic, scales, shifts):
    spec = pl.BlockSpec((_RB, D), lambda i: (i, 0))
    cspec = pl.BlockSpec((4, D), lambda i: (0, 0))
    return pl.pallas_call(
        _apply_body,
        grid=(_NB,),
        in_specs=[spec] * 4 + [cspec] * 2,
        out_specs=[spec] * 4,
        out_shape=[jax.ShapeDtypeStruct((NU, D), _f32)] * 4,
    )(xuh, xih, xuc, xic, scales, shifts)


def _main_body(t, bxu_h, bxu_c, bxi_h, bxi_c, m_uh, m_uc, m_ih, m_ic, eu, ei,
               Wu_eh, Wu_ec, Wi_eh, Wi_ec, Wz_u, Wh_u, Wc_u, Wz_i, Wh_i, Wc_i,
               zu_enc, zi_enc, xu_th, xu_tc, xi_th, xi_tc):
    decay = jnp.exp(-t[0, 0])

    def side(bxh, bxc, mh, mc, e, Weh, Wec, Wz, Wh, Wc, zenc, xth, xtc):
        ev = e[...]
        xmh = jnp.tanh(jnp.dot(decay * bxh[...] + mh[...], Weh[...],
                               preferred_element_type=_f32)) + ev
        xmc = jnp.tanh(jnp.dot(decay * bxc[...] + mc[...], Wec[...],
                               preferred_element_type=_f32)) + ev

        def two(Wr):
            return (jnp.dot(xmh, Wr[pl.ds(0, D), :],
                            preferred_element_type=_f32) +
                    jnp.dot(xmc, Wr[pl.ds(D, D), :],
                            preferred_element_type=_f32))

        zenc[:, pl.ds(0, D)] = jnp.tanh(two(Wz))
        zenc[:, pl.ds(D, D)] = ev
        xth[...] = jnp.tanh(two(Wh))
        xtc[...] = jnp.tanh(two(Wc))

    side(bxu_h, bxu_c, m_uh, m_uc, eu, Wu_eh, Wu_ec, Wz_u, Wh_u, Wc_u,
         zu_enc, xu_th, xu_tc)
    side(bxi_h, bxi_c, m_ih, m_ic, ei, Wi_eh, Wi_ec, Wz_i, Wh_i, Wc_i,
         zi_enc, xi_th, xi_tc)


def _tc_main(t2, bxu_h, bxu_c, bxi_h, bxi_c, m_uh, m_uc, m_ih, m_ic, eu, ei,
             Wu_eh, Wu_ec, Wi_eh, Wi_ec, Wz_u, Wh_u, Wc_u, Wz_i, Wh_i, Wc_i):
    spec = pl.BlockSpec((_RB, D), lambda i: (i, 0))
    spec2 = pl.BlockSpec((_RB, 2 * D), lambda i: (i, 0))
    w1 = pl.BlockSpec((D, D), lambda i: (0, 0))
    w2 = pl.BlockSpec((2 * D, D), lambda i: (0, 0))
    tspec = pl.BlockSpec((1, 1), lambda i: (0, 0))
    return pl.pallas_call(
        _main_body,
        grid=(_NB,),
        in_specs=[tspec] + [spec] * 10 + [w1] * 4 + [w2] * 6,
        out_specs=[spec2, spec2, spec, spec, spec, spec],
        out_shape=[
            jax.ShapeDtypeStruct((NU, 2 * D), _f32),
            jax.ShapeDtypeStruct((NI, 2 * D), _f32),
            jax.ShapeDtypeStruct((NU, D), _f32),
            jax.ShapeDtypeStruct((NU, D), _f32),
            jax.ShapeDtypeStruct((NI, D), _f32),
            jax.ShapeDtypeStruct((NI, D), _f32),
        ],
    )(t2, bxu_h, bxu_c, bxi_h, bxi_c, m_uh, m_uc, m_ih, m_ic, eu, ei,
      Wu_eh, Wu_ec, Wi_eh, Wi_ec, Wz_u, Wh_u, Wc_u, Wz_i, Wh_i, Wc_i)


def _loss_body(zup, zip_, zun, zin, Wpu, Wpi, o):
    pu = jnp.dot(zup[...], Wpu[...], preferred_element_type=_f32)
    pi = jnp.dot(zip_[...], Wpi[...], preferred_element_type=_f32)
    cols = [jnp.sum(pu * pi, 1, keepdims=True)]
    for k in range(K):
        pik = jnp.dot(zin[:, pl.ds(k * 2 * D, 2 * D)], Wpi[...],
                      preferred_element_type=_f32)
        cols.append(jnp.sum(pu * pik, 1, keepdims=True))
    for k in range(K):
        puk = jnp.dot(zun[:, pl.ds(k * 2 * D, 2 * D)], Wpu[...],
                      preferred_element_type=_f32)
        cols.append(jnp.sum(puk * pi, 1, keepdims=True))
    scores = jnp.concatenate(cols, axis=1)
    m = jnp.max(scores, 1, keepdims=True)
    lse = jnp.log(jnp.sum(jnp.exp(scores - m), 1, keepdims=True)) + m
    loss = jnp.mean(lse[:, 0] - scores[:, 0])
    o[...] = jnp.broadcast_to(loss, (8, 128))


def _tc_loss(zu_pos, zi_pos, zu_neg2, zi_neg2, Wp_u, Wp_i):
    full = lambda s: pl.BlockSpec(s, lambda: (0, 0))
    return pl.pallas_call(
        _loss_body,
        in_specs=[full((B, 2 * D)), full((B, 2 * D)),
                  full((B, 2 * D * K)), full((B, 2 * D * K)),
                  full((2 * D, D)), full((2 * D, D))],
        out_specs=full((8, 128)),
        out_shape=jax.ShapeDtypeStruct((8, 128), _f32),
    )(zu_pos, zi_pos, zu_neg2, zi_neg2, Wp_u, Wp_i)


def _final_body(xu_th, xi_th, xu_tc, xi_tc, mu_h, mi_h, mu_c, mi_c,
                Wu_uh, Wi_uh, Wu_uc, Wi_uc, ou_h, oi_h, ou_c, oi_c):
    for x, m, w, o in ((xu_th, mu_h, Wu_uh, ou_h), (xi_th, mi_h, Wi_uh, oi_h),
                       (xu_tc, mu_c, Wu_uc, ou_c), (xi_tc, mi_c, Wi_uc, oi_c)):
        o[...] = x[...] + jnp.tanh(
            jnp.dot(m[...], w[...], preferred_element_type=_f32))


def _tc_final(xu_th, xi_th, xu_tc, xi_tc, mu_h, mi_h, mu_c, mi_c,
              Wu_uh, Wi_uh, Wu_uc, Wi_uc):
    spec = pl.BlockSpec((_RB, D), lambda i: (i, 0))
    w1 = pl.BlockSpec((D, D), lambda i: (0, 0))
    return pl.pallas_call(
        _final_body,
        grid=(_NB,),
        in_specs=[spec] * 8 + [w1] * 4,
        out_specs=[spec] * 4,
        out_shape=[jax.ShapeDtypeStruct((NU, D), _f32)] * 4,
    )(xu_th, xi_th, xu_tc, xi_tc, mu_h, mi_h, mu_c, mi_c,
      Wu_uh, Wi_uh, Wu_uc, Wi_uc)


def kernel(t_diff, adj_his, adj_ctx, adj_tgt_i2u, adj_tgt_u2i, tgt_u, tgt_i,
           tgt_u_neg, tgt_i_neg, xu_in_his, xi_in_his, xu_in_ctx, xi_in_ctx,
           embeds_u, embeds_i, g_uh, b_uh, g_ih, b_ih, g_uc, b_uc, g_ic, b_ic,
           Wu_eh, Wi_eh, Wu_ec, Wi_ec, Wu_uh, Wi_uh, Wu_uc, Wi_uc,
           Wz_u, Wh_u, Wc_u, Wz_i, Wh_i, Wc_i, Wp_u, Wp_i):
    stats = _tc_stats(xu_in_his, xi_in_his, xu_in_ctx, xi_in_ctx)
    sums = stats[0::2, :] / NU
    sqs = stats[1::2, :] / NU
    var = sqs - sums * sums
    g = jnp.stack([g_uh, g_ih, g_uc, g_ic])
    bb = jnp.stack([b_uh, b_ih, b_uc, b_ic])
    scales = g / jnp.sqrt(var + 1e-5)
    shifts = bb - sums * scales
    bxu_h, bxi_h, bxu_c, bxi_c = _tc_apply(
        xu_in_his, xi_in_his, xu_in_ctx, xi_in_ctx, scales, shifts)

    pad = E_PAD - E
    padsrc = (jnp.arange(pad, dtype=_i32) * 61) % NU
    padbad = jnp.full((pad,), 2 * NU, _i32)

    def prep(row):
        src = jnp.concatenate([row, padsrc]).reshape(-1, 128)
        dst = jnp.concatenate([row, padbad]).reshape(-1, 128)
        return src, dst

    s_h_s, s_h_d = prep(adj_his[0])
    d_h_s, d_h_d = prep(adj_his[1])
    s_c_s, s_c_d = prep(adj_ctx[0])
    d_c_s, d_c_d = prep(adj_ctx[1])
    zrows = jnp.zeros((STRIPE, D), _f32)

    m_uh, m_ih, m_uc, m_ic = _sc_msgs(
        s_h_s, s_h_d, d_h_s, d_h_d, s_c_s, s_c_d, d_c_s, d_c_d,
        bxi_h, bxu_h, bxi_c, bxu_c, zrows)

    t2 = t_diff.reshape(1, 1)
    zu_enc, zi_enc, xu_th, xu_tc, xi_th, xi_tc = _tc_main(
        t2, bxu_h, bxu_c, bxi_h, bxi_c, m_uh, m_uc, m_ih, m_ic,
        embeds_u, embeds_i, Wu_eh, Wu_ec, Wi_eh, Wi_ec,
        Wz_u, Wh_u, Wc_u, Wz_i, Wh_i, Wc_i)

    zu_pos, zi_pos, zu_neg, zi_neg, mu_h, mi_h, mu_c, mi_c = _sc_targets(
        zu_enc, zi_enc, xu_th, xi_th, xu_tc, xi_tc,
        tgt_u.reshape(-1, 128), tgt_i.reshape(-1, 128),
        tgt_u_neg.reshape(-1, 128), tgt_i_neg.reshape(-1, 128),
        adj_tgt_i2u[0].reshape(-1, 128), adj_tgt_i2u[1].reshape(-1, 128),
        adj_tgt_u2i[0].reshape(-1, 128), adj_tgt_u2i[1].reshape(-1, 128),
        zrows)

    lossmat = _tc_loss(zu_pos, zi_pos,
                       zu_neg.reshape(B, 2 * D * K),
                       zi_neg.reshape(B, 2 * D * K), Wp_u, Wp_i)
    loss = lossmat[0, 0]

    ou_h, oi_h, ou_c, oi_c = _tc_final(
        xu_th, xi_th, xu_tc, xi_tc, mu_h, mi_h, mu_c, mi_c,
        Wu_uh, Wi_uh, Wu_uc, Wi_uc)

    return (loss, zu_pos, zi_enc, ou_h, oi_h, ou_c, oi_c)


# speedup vs baseline: 5.7413x; 1.1133x over previous
import functools

import jax
import jax.numpy as jnp
from jax import lax
from jax.experimental import pallas as pl
from jax.experimental.pallas import tpu as pltpu
from jax.experimental.pallas import tpu_sc as plsc

NU = 50000
NI = 50000
D = 64
E = 800000
B = 4096
K = 5

NC = 2
NS = 16
HALF = NU // NC
ACC_ROWS = 25088
STRIPE = 1568
LAST_STRIPE = HALF - (NS - 1) * STRIPE
E_PAD = 802816
NCHUNK = 49

_f32 = jnp.float32
_i32 = jnp.int32


HD = D // 2
ACC2 = 50048
WSTRIPE = 3128
WLAST = NU - (NS - 1) * WSTRIPE


def _sc_edge_phase(srcr, dstr, tab3, outr, zrows, src_v, dst_v, rows,
                   acc, gsem, ssem, cid, sid):
    tab = tab3.at[cid]

    pltpu.sync_copy(zrows, acc.at[pl.ds(sid * WSTRIPE, WSTRIPE)])
    plsc.subcore_barrier()

    def chunk(i, carry):
        erow = sid * (NCHUNK * 8) + i * 8
        pltpu.sync_copy(srcr.at[pl.ds(erow, 8)], src_v)
        pltpu.sync_copy(dstr.at[pl.ds(erow, 8)], dst_v)
        gd = [None] * 8
        sd = [None] * 8
        gd[0] = pltpu.async_copy(tab.at[src_v.at[0]],
                                 rows.at[pl.ds(0, 128)], gsem)
        for j in range(8):
            if j + 1 < 8:
                if j >= 3:
                    sd[j - 3].wait()
                gd[j + 1] = pltpu.async_copy(
                    tab.at[src_v.at[j + 1]],
                    rows.at[pl.ds(((j + 1) % 4) * 128, 128)], gsem)
            gd[j].wait()
            sd[j] = pltpu.async_copy(rows.at[pl.ds((j % 4) * 128, 128)],
                                     acc.at[dst_v.at[j]], ssem, add=True)
        for j in range(4, 8):
            sd[j].wait()
        return carry

    lax.fori_loop(0, NCHUNK, chunk, 0)
    plsc.subcore_barrier()

    for cc in range(NC):
        @pl.when((cid == cc) & (sid < NS - 1))
        def _(cc=cc):
            pltpu.sync_copy(
                acc.at[pl.ds(sid * WSTRIPE, WSTRIPE)],
                outr.at[pl.ds(sid * WSTRIPE, WSTRIPE),
                        pl.ds(cc * HD, HD)])

        @pl.when((cid == cc) & (sid == NS - 1))
        def _(cc=cc):
            pltpu.sync_copy(
                acc.at[pl.ds((NS - 1) * WSTRIPE, WLAST)],
                outr.at[pl.ds((NS - 1) * WSTRIPE, WLAST),
                        pl.ds(cc * HD, HD)])


def _sc_msgs_body(s_h_s, s_h_d, d_h_s, d_h_d, s_c_s, s_c_d, d_c_s, d_c_d,
                  tb_ih, tb_uh, tb_ic, tb_uc, zrows,
                  o_uh, o_ih, o_uc, o_ic,
                  src_v, dst_v, rows, acc, gsem, ssem):
    cid = lax.axis_index("c")
    sid = lax.axis_index("s")
    phases = (
        (d_h_s, s_h_d, tb_ih, o_uh),
        (s_h_s, d_h_d, tb_uh, o_ih),
        (d_c_s, s_c_d, tb_ic, o_uc),
        (s_c_s, d_c_d, tb_uc, o_ic),
    )
    for srcr, dstr, tab3, outr in phases:
        _sc_edge_phase(srcr, dstr, tab3, outr, zrows, src_v, dst_v,
                       rows, acc, gsem, ssem, cid, sid)


_sc_msgs = functools.partial(
    pl.kernel,
    out_type=tuple(jax.ShapeDtypeStruct((NU, D), _f32) for _ in range(4)),
    mesh=plsc.VectorSubcoreMesh(core_axis_name="c", subcore_axis_name="s"),
    scratch_types=[
        pltpu.VMEM((8, 128), _i32),
        pltpu.VMEM((8, 128), _i32),
        pltpu.VMEM((512, HD), _f32),
        pltpu.VMEM_SHARED((ACC2, HD), _f32),
        pltpu.SemaphoreType.DMA,
        pltpu.SemaphoreType.DMA,
    ],
    compiler_params=pltpu.CompilerParams(use_tc_tiling_on_sc=False),
)(_sc_msgs_body)


def _sc_targets_body(zu_enc, zi_enc, xu_th, xi_th, xu_tc, xi_tc,
                     tgtu, tgti, tgtun, tgtin,
                     i2u_s, i2u_d, u2i_s, u2i_d, zrows,
                     zu_pos, zi_pos, zu_neg, zi_neg,
                     mu_h, mi_h, mu_c, mi_c,
                     idx_v, rows128, rows64, ldst, acc, gsem, ssem):
    cid = lax.axis_index("c")
    sid = lax.axis_index("s")
    wid = sid * NC + cid

    for idx2d, tab, outr, n_per_w in (
            (tgtu, zu_enc, zu_pos, 1),
            (tgti, zi_enc, zi_pos, 1),
            (tgtun, zu_enc, zu_neg, 5),
            (tgtin, zi_enc, zi_neg, 5),
    ):
        for q in range(n_per_w):
            crow = wid * n_per_w + q
            pltpu.sync_copy(idx2d.at[pl.ds(crow, 1)], idx_v.at[pl.ds(0, 1)])
            pltpu.async_copy(tab.at[idx_v.at[0]], rows128, gsem).wait()
            pltpu.sync_copy(rows128, outr.at[pl.ds(crow * 128, 128)])

    base_row = cid * HALF
    trash = HALF + 8 + sid * 4
    for src2d, dst2d, tab, outr in (
            (i2u_s, i2u_d, xi_th, mu_h),
            (u2i_s, u2i_d, xu_th, mi_h),
            (i2u_s, i2u_d, xi_tc, mu_c),
            (u2i_s, u2i_d, xu_tc, mi_c),
    ):
        pltpu.sync_copy(zrows, acc.at[pl.ds(sid * STRIPE, STRIPE)])
        plsc.subcore_barrier()
        for q in range(2):
            crow = sid * 2 + q
            pltpu.sync_copy(src2d.at[pl.ds(crow, 1)], idx_v.at[pl.ds(0, 1)])
            pltpu.sync_copy(dst2d.at[pl.ds(crow, 1)], idx_v.at[pl.ds(1, 1)])
            dsc = pltpu.async_copy(tab.at[idx_v.at[0]], rows64, gsem)
            for t in range(8):
                v = idx_v[1, pl.ds(t * 16, 16)]
                tl = v - base_row
                m = (tl >= 0) & (tl < HALF)
                ldst[0, pl.ds(t * 16, 16)] = jnp.where(m, tl, trash)
            dsc.wait()
            pltpu.async_copy(rows64, acc.at[ldst.at[0]], ssem,
                             add=True).wait()
        plsc.subcore_barrier()

        @pl.when(sid < NS - 1)
        def _():
            pltpu.sync_copy(acc.at[pl.ds(sid * STRIPE, STRIPE)],
                            outr.at[pl.ds(base_row + sid * STRIPE, STRIPE)])

        @pl.when(sid == NS - 1)
        def _():
            pltpu.sync_copy(acc.at[pl.ds((NS - 1) * STRIPE, LAST_STRIPE)],
                            outr.at[pl.ds(base_row + (NS - 1) * STRIPE,
                                          LAST_STRIPE)])


_sc_targets = functools.partial(
    pl.kernel,
    out_type=(
        jax.ShapeDtypeStruct((B, 2 * D), _f32),
        jax.ShapeDtypeStruct((B, 2 * D), _f32),
        jax.ShapeDtypeStruct((B * K, 2 * D), _f32),
        jax.ShapeDtypeStruct((B * K, 2 * D), _f32),
        jax.ShapeDtypeStruct((NU, D), _f32),
        jax.ShapeDtypeStruct((NI, D), _f32),
        jax.ShapeDtypeStruct((NU, D), _f32),
        jax.ShapeDtypeStruct((NI, D), _f32),
    ),
    mesh=plsc.VectorSubcoreMesh(core_axis_name="c", subcore_axis_name="s"),
    scratch_types=[
        pltpu.VMEM((8, 128), _i32),
        pltpu.VMEM((128, 2 * D), _f32),
        pltpu.VMEM((128, D), _f32),
        pltpu.VMEM((8, 128), _i32),
        pltpu.VMEM_SHARED((ACC_ROWS, D), _f32),
        pltpu.SemaphoreType.DMA,
        pltpu.SemaphoreType.DMA,
    ],
    compiler_params=pltpu.CompilerParams(use_tc_tiling_on_sc=False),
)(_sc_targets_body)


_RB = 1000
_NB = NU // _RB


def _stats_body(a, b, c, d, o):
    i = pl.program_id(0)

    @pl.when(i == 0)
    def _():
        o[...] = jnp.zeros_like(o)

    for r, x in enumerate((a, b, c, d)):
        xv = x[...]
        o[pl.ds(2 * r, 1), :] += jnp.sum(xv, 0, keepdims=True)
        o[pl.ds(2 * r + 1, 1), :] += jnp.sum(xv * xv, 0, keepdims=True)


def _tc_stats(xuh, xih, xuc, xic):
    spec = pl.BlockSpec((_RB, D), lambda i: (i, 0))
    return pl.pallas_call(
        _stats_body,
        grid=(_NB,),
        in_specs=[spec] * 4,
        out_specs=pl.BlockSpec((8, D), lambda i: (0, 0)),
        out_shape=jax.ShapeDtypeStruct((8, D), _f32),
    )(xuh, xih, xuc, xic)


def _apply_body(a, b, c, d, sc, sh, oa, ob, oc, od):
    for r, (x, o) in enumerate(((a, oa), (b, ob), (c, oc), (d, od))):
        o[...] = x[...] * sc[pl.ds(r, 1), :] + sh[pl.ds(r, 1), :]


def _tc_apply(xuh, xih, xuc, xic, scales, shifts):
    spec = pl.BlockSpec((_RB, D), lambda i: (i, 0))
    cspec = pl.BlockSpec((4, D), lambda i: (0, 0))
    return pl.pallas_call(
        _apply_body,
        grid=(_NB,),
        in_specs=[spec] * 4 + [cspec] * 2,
        out_specs=[spec] * 4,
        out_shape=[jax.ShapeDtypeStruct((NU, D), _f32)] * 4,
    )(xuh, xih, xuc, xic, scales, shifts)


def _main_body(t, bxu_h, bxu_c, bxi_h, bxi_c, m_uh, m_uc, m_ih, m_ic, eu, ei,
               Wu_eh, Wu_ec, Wi_eh, Wi_ec, Wz_u, Wh_u, Wc_u, Wz_i, Wh_i, Wc_i,
               zu_enc, zi_enc, xu_th, xu_tc, xi_th, xi_tc):
    decay = jnp.exp(-t[0, 0])

    def side(bxh, bxc, mh, mc, e, Weh, Wec, Wz, Wh, Wc, zenc, xth, xtc):
        ev = e[...]
        xmh = jnp.tanh(jnp.dot(decay * bxh[...] + mh[...], Weh[...],
                               preferred_element_type=_f32)) + ev
        xmc = jnp.tanh(jnp.dot(decay * bxc[...] + mc[...], Wec[...],
                               preferred_element_type=_f32)) + ev

        def two(Wr):
            return (jnp.dot(xmh, Wr[pl.ds(0, D), :],
                            preferred_element_type=_f32) +
                    jnp.dot(xmc, Wr[pl.ds(D, D), :],
                            preferred_element_type=_f32))

        zenc[:, pl.ds(0, D)] = jnp.tanh(two(Wz))
        zenc[:, pl.ds(D, D)] = ev
        xth[...] = jnp.tanh(two(Wh))
        xtc[...] = jnp.tanh(two(Wc))

    side(bxu_h, bxu_c, m_uh, m_uc, eu, Wu_eh, Wu_ec, Wz_u, Wh_u, Wc_u,
         zu_enc, xu_th, xu_tc)
    side(bxi_h, bxi_c, m_ih, m_ic, ei, Wi_eh, Wi_ec, Wz_i, Wh_i, Wc_i,
         zi_enc, xi_th, xi_tc)


def _tc_main(t2, bxu_h, bxu_c, bxi_h, bxi_c, m_uh, m_uc, m_ih, m_ic, eu, ei,
             Wu_eh, Wu_ec, Wi_eh, Wi_ec, Wz_u, Wh_u, Wc_u, Wz_i, Wh_i, Wc_i):
    spec = pl.BlockSpec((_RB, D), lambda i: (i, 0))
    spec2 = pl.BlockSpec((_RB, 2 * D), lambda i: (i, 0))
    w1 = pl.BlockSpec((D, D), lambda i: (0, 0))
    w2 = pl.BlockSpec((2 * D, D), lambda i: (0, 0))
    tspec = pl.BlockSpec((1, 1), lambda i: (0, 0))
    return pl.pallas_call(
        _main_body,
        grid=(_NB,),
        in_specs=[tspec] + [spec] * 10 + [w1] * 4 + [w2] * 6,
        out_specs=[spec2, spec2, spec, spec, spec, spec],
        out_shape=[
            jax.ShapeDtypeStruct((NU, 2 * D), _f32),
            jax.ShapeDtypeStruct((NI, 2 * D), _f32),
            jax.ShapeDtypeStruct((NU, D), _f32),
            jax.ShapeDtypeStruct((NU, D), _f32),
            jax.ShapeDtypeStruct((NI, D), _f32),
            jax.ShapeDtypeStruct((NI, D), _f32),
        ],
    )(t2, bxu_h, bxu_c, bxi_h, bxi_c, m_uh, m_uc, m_ih, m_ic, eu, ei,
      Wu_eh, Wu_ec, Wi_eh, Wi_ec, Wz_u, Wh_u, Wc_u, Wz_i, Wh_i, Wc_i)


def _loss_body(zup, zip_, zun, zin, Wpu, Wpi, o):
    pu = jnp.dot(zup[...], Wpu[...], preferred_element_type=_f32)
    pi = jnp.dot(zip_[...], Wpi[...], preferred_element_type=_f32)
    cols = [jnp.sum(pu * pi, 1, keepdims=True)]
    for k in range(K):
        pik = jnp.dot(zin[:, pl.ds(k * 2 * D, 2 * D)], Wpi[...],
                      preferred_element_type=_f32)
        cols.append(jnp.sum(pu * pik, 1, keepdims=True))
    for k in range(K):
        puk = jnp.dot(zun[:, pl.ds(k * 2 * D, 2 * D)], Wpu[...],
                      preferred_element_type=_f32)
        cols.append(jnp.sum(puk * pi, 1, keepdims=True))
    scores = jnp.concatenate(cols, axis=1)
    m = jnp.max(scores, 1, keepdims=True)
    lse = jnp.log(jnp.sum(jnp.exp(scores - m), 1, keepdims=True)) + m
    loss = jnp.mean(lse[:, 0] - scores[:, 0])
    o[...] = jnp.broadcast_to(loss, (8, 128))


def _tc_loss(zu_pos, zi_pos, zu_neg2, zi_neg2, Wp_u, Wp_i):
    full = lambda s: pl.BlockSpec(s, lambda: (0, 0))
    return pl.pallas_call(
        _loss_body,
        in_specs=[full((B, 2 * D)), full((B, 2 * D)),
                  full((B, 2 * D * K)), full((B, 2 * D * K)),
                  full((2 * D, D)), full((2 * D, D))],
        out_specs=full((8, 128)),
        out_shape=jax.ShapeDtypeStruct((8, 128), _f32),
    )(zu_pos, zi_pos, zu_neg2, zi_neg2, Wp_u, Wp_i)


def _final_body(xu_th, xi_th, xu_tc, xi_tc, mu_h, mi_h, mu_c, mi_c,
                Wu_uh, Wi_uh, Wu_uc, Wi_uc, ou_h, oi_h, ou_c, oi_c):
    for x, m, w, o in ((xu_th, mu_h, Wu_uh, ou_h), (xi_th, mi_h, Wi_uh, oi_h),
                       (xu_tc, mu_c, Wu_uc, ou_c), (xi_tc, mi_c, Wi_uc, oi_c)):
        o[...] = x[...] + jnp.tanh(
            jnp.dot(m[...], w[...], preferred_element_type=_f32))


def _tc_final(xu_th, xi_th, xu_tc, xi_tc, mu_h, mi_h, mu_c, mi_c,
              Wu_uh, Wi_uh, Wu_uc, Wi_uc):
    spec = pl.BlockSpec((_RB, D), lambda i: (i, 0))
    w1 = pl.BlockSpec((D, D), lambda i: (0, 0))
    return pl.pallas_call(
        _final_body,
        grid=(_NB,),
        in_specs=[spec] * 8 + [w1] * 4,
        out_specs=[spec] * 4,
        out_shape=[jax.ShapeDtypeStruct((NU, D), _f32)] * 4,
    )(xu_th, xi_th, xu_tc, xi_tc, mu_h, mi_h, mu_c, mi_c,
      Wu_uh, Wi_uh, Wu_uc, Wi_uc)


def kernel(t_diff, adj_his, adj_ctx, adj_tgt_i2u, adj_tgt_u2i, tgt_u, tgt_i,
           tgt_u_neg, tgt_i_neg, xu_in_his, xi_in_his, xu_in_ctx, xi_in_ctx,
           embeds_u, embeds_i, g_uh, b_uh, g_ih, b_ih, g_uc, b_uc, g_ic, b_ic,
           Wu_eh, Wi_eh, Wu_ec, Wi_ec, Wu_uh, Wi_uh, Wu_uc, Wi_uc,
           Wz_u, Wh_u, Wc_u, Wz_i, Wh_i, Wc_i, Wp_u, Wp_i):
    stats = _tc_stats(xu_in_his, xi_in_his, xu_in_ctx, xi_in_ctx)
    sums = stats[0::2, :] / NU
    sqs = stats[1::2, :] / NU
    var = sqs - sums * sums
    g = jnp.stack([g_uh, g_ih, g_uc, g_ic])
    bb = jnp.stack([b_uh, b_ih, b_uc, b_ic])
    scales = g / jnp.sqrt(var + 1e-5)
    shifts = bb - sums * scales
    bxu_h, bxi_h, bxu_c, bxi_c = _tc_apply(
        xu_in_his, xi_in_his, xu_in_ctx, xi_in_ctx, scales, shifts)

    pad = E_PAD - E
    padsrc = (jnp.arange(pad, dtype=_i32) * 61) % NU
    padbad = NU + (jnp.arange(pad, dtype=_i32) % (ACC2 - NU))

    def prep(row):
        src = jnp.concatenate([row, padsrc]).reshape(-1, 128)
        dst = jnp.concatenate([row, padbad]).reshape(-1, 128)
        return src, dst

    s_h_s, s_h_d = prep(adj_his[0])
    d_h_s, d_h_d = prep(adj_his[1])
    s_c_s, s_c_d = prep(adj_ctx[0])
    d_c_s, d_c_d = prep(adj_ctx[1])
    zrows = jnp.zeros((STRIPE, D), _f32)
    zrows32 = jnp.zeros((WSTRIPE, HD), _f32)

    def colsplit(x):
        return jnp.stack([x[:, :HD], x[:, HD:]])

    m_uh, m_ih, m_uc, m_ic = _sc_msgs(
        s_h_s, s_h_d, d_h_s, d_h_d, s_c_s, s_c_d, d_c_s, d_c_d,
        colsplit(bxi_h), colsplit(bxu_h), colsplit(bxi_c), colsplit(bxu_c),
        zrows32)

    t2 = t_diff.reshape(1, 1)
    zu_enc, zi_enc, xu_th, xu_tc, xi_th, xi_tc = _tc_main(
        t2, bxu_h, bxu_c, bxi_h, bxi_c, m_uh, m_uc, m_ih, m_ic,
        embeds_u, embeds_i, Wu_eh, Wu_ec, Wi_eh, Wi_ec,
        Wz_u, Wh_u, Wc_u, Wz_i, Wh_i, Wc_i)

    zu_pos, zi_pos, zu_neg, zi_neg, mu_h, mi_h, mu_c, mi_c = _sc_targets(
        zu_enc, zi_enc, xu_th, xi_th, xu_tc, xi_tc,
        tgt_u.reshape(-1, 128), tgt_i.reshape(-1, 128),
        tgt_u_neg.reshape(-1, 128), tgt_i_neg.reshape(-1, 128),
        adj_tgt_i2u[0].reshape(-1, 128), adj_tgt_i2u[1].reshape(-1, 128),
        adj_tgt_u2i[0].reshape(-1, 128), adj_tgt_u2i[1].reshape(-1, 128),
        zrows)

    lossmat = _tc_loss(zu_pos, zi_pos,
                       zu_neg.reshape(B, 2 * D * K),
                       zi_neg.reshape(B, 2 * D * K), Wp_u, Wp_i)
    loss = lossmat[0, 0]

    ou_h, oi_h, ou_c, oi_c = _tc_final(
        xu_th, xi_th, xu_tc, xi_tc, mu_h, mi_h, mu_c, mi_c,
        Wu_uh, Wi_uh, Wu_uc, Wi_uc)

    return (loss, zu_pos, zi_enc, ou_h, oi_h, ou_c, oi_c)


# speedup vs baseline: 6.5594x; 1.1425x over previous
import functools

import jax
import jax.numpy as jnp
from jax import lax
from jax.experimental import pallas as pl
from jax.experimental.pallas import tpu as pltpu
from jax.experimental.pallas import tpu_sc as plsc

NU = 50000
NI = 50000
D = 64
E = 800000
B = 4096
K = 5

NC = 2
NS = 16
HALF = NU // NC
ACC_ROWS = 25088
STRIPE = 1568
LAST_STRIPE = HALF - (NS - 1) * STRIPE
E_PAD = 802816
NCHUNK = 49

_f32 = jnp.float32
_i32 = jnp.int32


HD = D // 2
ACC2 = 50048
WSTRIPE = 3128
WLAST = NU - (NS - 1) * WSTRIPE


def _sc_edge_phase(srcr, dstr, tab, outr, degr, do_deg, zrows, zdeg,
                   src_v, dst_v, rows, ones_v, acc, dacc, gsem, ssem,
                   cid, sid):
    pltpu.sync_copy(zrows, acc.at[pl.ds(sid * WSTRIPE, WSTRIPE)])
    pltpu.sync_copy(zdeg, dacc.at[pl.ds(sid * WSTRIPE, WSTRIPE)])
    plsc.subcore_barrier()

    for cc in range(NC):
        @pl.when(cid == cc)
        def _(cc=cc):
            tabh = tab.at[cc]
            my_deg = do_deg == cc

            def chunk(i, carry):
                erow = sid * (NCHUNK * 8) + i * 8
                pltpu.sync_copy(srcr.at[pl.ds(erow, 8)], src_v)
                pltpu.sync_copy(dstr.at[pl.ds(erow, 8)], dst_v)
                gd = [None] * 8
                sd = [None] * 8
                gd[0] = pltpu.async_copy(tabh.at[src_v.at[0]],
                                         rows.at[pl.ds(0, 128)], gsem)
                for j in range(8):
                    if j + 1 < 8:
                        if j >= 4:
                            sd[j - 4].wait()
                        gd[j + 1] = pltpu.async_copy(
                            tabh.at[src_v.at[j + 1]],
                            rows.at[pl.ds(((j + 1) % 5) * 128, 128)], gsem)
                    gd[j].wait()
                    sd[j] = pltpu.async_copy(
                        rows.at[pl.ds((j % 5) * 128, 128)],
                        acc.at[dst_v.at[j]], ssem, add=True)
                    if my_deg:
                        pltpu.sync_copy(ones_v, dacc.at[dst_v.at[j]],
                                        add=True)
                for j in range(3, 8):
                    sd[j].wait()
                return carry

            lax.fori_loop(0, NCHUNK, chunk, 0)

    plsc.subcore_barrier()

    nrows = [WSTRIPE, WLAST]
    for cc in range(NC):
        for last in range(2):
            sel = (sid == NS - 1) if last else (sid < NS - 1)
            r0c = (NS - 1) * WSTRIPE if last else None

            @pl.when((cid == cc) & sel)
            def _(cc=cc, n=nrows[last], last=last):
                r0 = (NS - 1) * WSTRIPE if last else sid * WSTRIPE
                pltpu.sync_copy(acc.at[pl.ds(r0, n)],
                                outr.at[pl.ds(r0, n), pl.ds(cc * HD, HD)])

            if do_deg == cc:
                @pl.when((cid == cc) & sel)
                def _(n=nrows[last], last=last):
                    r0 = (NS - 1) * WSTRIPE if last else sid * WSTRIPE
                    pltpu.sync_copy(dacc.at[pl.ds(r0, n)],
                                    degr.at[pl.ds(r0, n)])


def _sc_msgs_body(s_h_s, s_h_d, d_h_s, d_h_d, s_c_s, s_c_d, d_c_s, d_c_d,
                  r_ih, r_uh, r_ic, r_uc, zrows, zdeg,
                  o_uh, o_ih, o_uc, o_ic, dg_uh, dg_ih, dg_uc, dg_ic,
                  tb_ih, tb_uh, tb_ic, tb_uc,
                  src_v, dst_v, rows, ones_v, acc, dacc, gsem, ssem):
    cid = lax.axis_index("c")
    sid = lax.axis_index("s")
    for t in range(8):
        ones_v[pl.ds(t * 16, 16)] = jnp.full((16,), 1.0, _f32)

    for raw, tb in ((r_ih, tb_ih), (r_uh, tb_uh), (r_ic, tb_ic),
                    (r_uc, tb_uc)):
        for cc in range(NC):
            @pl.when(cid == cc)
            def _(raw=raw, tb=tb, cc=cc):
                base = sid * WSTRIPE

                def cp(r0, n):
                    pltpu.sync_copy(
                        raw.at[pl.ds(r0, n), pl.ds(cc * HD, HD)],
                        rows.at[pl.ds(0, n)])
                    pltpu.sync_copy(rows.at[pl.ds(0, n)],
                                    tb.at[cc].at[pl.ds(r0, n)])

                def q4(q, carry):
                    cp(base + q * 640, 640)
                    return carry

                lax.fori_loop(0, 4, q4, 0)

                @pl.when(sid < NS - 1)
                def _():
                    cp(base + 2560, WSTRIPE - 2560)

                @pl.when(sid == NS - 1)
                def _():
                    cp(base + 2560, WLAST - 2560)
    plsc.subcore_barrier()

    phases = (
        (d_h_s, s_h_d, tb_ih, o_uh, dg_uh, 0),
        (s_h_s, d_h_d, tb_uh, o_ih, dg_ih, 1),
        (d_c_s, s_c_d, tb_ic, o_uc, dg_uc, 0),
        (s_c_s, d_c_d, tb_uc, o_ic, dg_ic, 1),
    )
    for srcr, dstr, tab, outr, degr, do_deg in phases:
        _sc_edge_phase(srcr, dstr, tab, outr, degr, do_deg, zrows, zdeg,
                       src_v, dst_v, rows, ones_v, acc, dacc, gsem, ssem,
                       cid, sid)


_sc_msgs = functools.partial(
    pl.kernel,
    out_type=tuple(
        [jax.ShapeDtypeStruct((NU, D), _f32) for _ in range(4)]
        + [jax.ShapeDtypeStruct((NU,), _f32) for _ in range(4)]
        + [jax.ShapeDtypeStruct((NC, NU, HD), _f32) for _ in range(4)]),
    mesh=plsc.VectorSubcoreMesh(core_axis_name="c", subcore_axis_name="s"),
    scratch_types=[
        pltpu.VMEM((8, 128), _i32),
        pltpu.VMEM((8, 128), _i32),
        pltpu.VMEM((640, HD), _f32),
        pltpu.VMEM((128,), _f32),
        pltpu.VMEM_SHARED((ACC2, HD), _f32),
        pltpu.VMEM_SHARED((ACC2,), _f32),
        pltpu.SemaphoreType.DMA,
        pltpu.SemaphoreType.DMA,
    ],
    compiler_params=pltpu.CompilerParams(use_tc_tiling_on_sc=False),
)(_sc_msgs_body)


def _sc_targets_body(zu_enc, zi_enc, xu_th, xi_th, xu_tc, xi_tc,
                     tgtu, tgti, tgtun, tgtin,
                     i2u_s, i2u_d, u2i_s, u2i_d, zrows,
                     zu_pos, zi_pos, zu_neg, zi_neg,
                     mu_h, mi_h, mu_c, mi_c,
                     idx_v, rows128, rows64, ldst, acc, gsem, ssem):
    cid = lax.axis_index("c")
    sid = lax.axis_index("s")
    wid = sid * NC + cid

    for idx2d, tab, outr, n_per_w in (
            (tgtu, zu_enc, zu_pos, 1),
            (tgti, zi_enc, zi_pos, 1),
            (tgtun, zu_enc, zu_neg, 5),
            (tgtin, zi_enc, zi_neg, 5),
    ):
        for q in range(n_per_w):
            crow = wid * n_per_w + q
            pltpu.sync_copy(idx2d.at[pl.ds(crow, 1)], idx_v.at[pl.ds(0, 1)])
            pltpu.async_copy(tab.at[idx_v.at[0]], rows128, gsem).wait()
            pltpu.sync_copy(rows128, outr.at[pl.ds(crow * 128, 128)])

    base_row = cid * HALF
    trash = HALF + 8 + sid * 4
    for src2d, dst2d, tab, outr in (
            (i2u_s, i2u_d, xi_th, mu_h),
            (u2i_s, u2i_d, xu_th, mi_h),
            (i2u_s, i2u_d, xi_tc, mu_c),
            (u2i_s, u2i_d, xu_tc, mi_c),
    ):
        pltpu.sync_copy(zrows, acc.at[pl.ds(sid * STRIPE, STRIPE)])
        plsc.subcore_barrier()
        for q in range(2):
            crow = sid * 2 + q
            pltpu.sync_copy(src2d.at[pl.ds(crow, 1)], idx_v.at[pl.ds(0, 1)])
            pltpu.sync_copy(dst2d.at[pl.ds(crow, 1)], idx_v.at[pl.ds(1, 1)])
            dsc = pltpu.async_copy(tab.at[idx_v.at[0]], rows64, gsem)
            for t in range(8):
                v = idx_v[1, pl.ds(t * 16, 16)]
                tl = v - base_row
                m = (tl >= 0) & (tl < HALF)
                ldst[0, pl.ds(t * 16, 16)] = jnp.where(m, tl, trash)
            dsc.wait()
            pltpu.async_copy(rows64, acc.at[ldst.at[0]], ssem,
                             add=True).wait()
        plsc.subcore_barrier()

        @pl.when(sid < NS - 1)
        def _():
            pltpu.sync_copy(acc.at[pl.ds(sid * STRIPE, STRIPE)],
                            outr.at[pl.ds(base_row + sid * STRIPE, STRIPE)])

        @pl.when(sid == NS - 1)
        def _():
            pltpu.sync_copy(acc.at[pl.ds((NS - 1) * STRIPE, LAST_STRIPE)],
                            outr.at[pl.ds(base_row + (NS - 1) * STRIPE,
                                          LAST_STRIPE)])


_sc_targets = functools.partial(
    pl.kernel,
    out_type=(
        jax.ShapeDtypeStruct((B, 2 * D), _f32),
        jax.ShapeDtypeStruct((B, 2 * D), _f32),
        jax.ShapeDtypeStruct((B * K, 2 * D), _f32),
        jax.ShapeDtypeStruct((B * K, 2 * D), _f32),
        jax.ShapeDtypeStruct((NU, D), _f32),
        jax.ShapeDtypeStruct((NI, D), _f32),
        jax.ShapeDtypeStruct((NU, D), _f32),
        jax.ShapeDtypeStruct((NI, D), _f32),
    ),
    mesh=plsc.VectorSubcoreMesh(core_axis_name="c", subcore_axis_name="s"),
    scratch_types=[
        pltpu.VMEM((8, 128), _i32),
        pltpu.VMEM((128, 2 * D), _f32),
        pltpu.VMEM((128, D), _f32),
        pltpu.VMEM((8, 128), _i32),
        pltpu.VMEM_SHARED((ACC_ROWS, D), _f32),
        pltpu.SemaphoreType.DMA,
        pltpu.SemaphoreType.DMA,
    ],
    compiler_params=pltpu.CompilerParams(use_tc_tiling_on_sc=False),
)(_sc_targets_body)


_RB = 1000
_NB = NU // _RB


def _stats_body(a, b, c, d, o):
    i = pl.program_id(0)

    @pl.when(i == 0)
    def _():
        o[...] = jnp.zeros_like(o)

    for r, x in enumerate((a, b, c, d)):
        xv = x[...]
        o[pl.ds(2 * r, 1), :] += jnp.sum(xv, 0, keepdims=True)
        o[pl.ds(2 * r + 1, 1), :] += jnp.sum(xv * xv, 0, keepdims=True)


def _tc_stats(xuh, xih, xuc, xic):
    spec = pl.BlockSpec((_RB, D), lambda i: (i, 0))
    return pl.pallas_call(
        _stats_body,
        grid=(_NB,),
        in_specs=[spec] * 4,
        out_specs=pl.BlockSpec((8, D), lambda i: (0, 0)),
        out_shape=jax.ShapeDtypeStruct((8, D), _f32),
    )(xuh, xih, xuc, xic)


def _main_body(t, xu_h, xu_c, xi_h, xi_c, m_uh, m_uc, m_ih, m_ic, degs,
               eu, ei, scv, shv,
               Wu_eh, Wu_ec, Wi_eh, Wi_ec, Wz_u, Wh_u, Wc_u, Wz_i, Wh_i, Wc_i,
               zu_enc, zi_enc, xu_th, xu_tc, xi_th, xi_tc):
    decay = jnp.exp(-t[0, 0])
    sc = scv[...]
    sh = shv[...]
    dg = degs[...]

    def bn(x, r):
        return x[...] * sc[r:r + 1, :] + sh[r:r + 1, :]

    def evo(xraw, rx, m, rg, p, W, e):
        h = (decay * bn(xraw, rx) + m[...] * sc[rg:rg + 1, :]
             + dg[:, p:p + 1] * sh[rg:rg + 1, :])
        return jnp.tanh(jnp.dot(h, W[...], preferred_element_type=_f32)) + e

    def side(xrh, xrc, rxh, rxc, mh, mc, rgh, rgc, ph, pc, e,
             Weh, Wec, Wz, Wh, Wc, zenc, xth, xtc):
        ev = e[...]
        xmh = evo(xrh, rxh, mh, rgh, ph, Weh, ev)
        xmc = evo(xrc, rxc, mc, rgc, pc, Wec, ev)

        def two(Wr):
            return (jnp.dot(xmh, Wr[pl.ds(0, D), :],
                            preferred_element_type=_f32) +
                    jnp.dot(xmc, Wr[pl.ds(D, D), :],
                            preferred_element_type=_f32))

        zenc[:, pl.ds(0, D)] = jnp.tanh(two(Wz))
        zenc[:, pl.ds(D, D)] = ev
        xth[...] = jnp.tanh(two(Wh))
        xtc[...] = jnp.tanh(two(Wc))

    side(xu_h, xu_c, 0, 2, m_uh, m_uc, 1, 3, 0, 2, eu,
         Wu_eh, Wu_ec, Wz_u, Wh_u, Wc_u, zu_enc, xu_th, xu_tc)
    side(xi_h, xi_c, 1, 3, m_ih, m_ic, 0, 2, 1, 3, ei,
         Wi_eh, Wi_ec, Wz_i, Wh_i, Wc_i, zi_enc, xi_th, xi_tc)


def _tc_main(t2, xu_h, xu_c, xi_h, xi_c, m_uh, m_uc, m_ih, m_ic, degs,
             eu, ei, scales, shifts,
             Wu_eh, Wu_ec, Wi_eh, Wi_ec, Wz_u, Wh_u, Wc_u, Wz_i, Wh_i, Wc_i):
    spec = pl.BlockSpec((_RB, D), lambda i: (i, 0))
    spec2 = pl.BlockSpec((_RB, 2 * D), lambda i: (i, 0))
    dspec = pl.BlockSpec((_RB, 4), lambda i: (i, 0))
    w1 = pl.BlockSpec((D, D), lambda i: (0, 0))
    w2 = pl.BlockSpec((2 * D, D), lambda i: (0, 0))
    cspec = pl.BlockSpec((4, D), lambda i: (0, 0))
    tspec = pl.BlockSpec((1, 1), lambda i: (0, 0))
    return pl.pallas_call(
        _main_body,
        grid=(_NB,),
        in_specs=[tspec] + [spec] * 8 + [dspec] + [spec] * 2 + [cspec] * 2
                 + [w1] * 4 + [w2] * 6,
        out_specs=[spec2, spec2, spec, spec, spec, spec],
        out_shape=[
            jax.ShapeDtypeStruct((NU, 2 * D), _f32),
            jax.ShapeDtypeStruct((NI, 2 * D), _f32),
            jax.ShapeDtypeStruct((NU, D), _f32),
            jax.ShapeDtypeStruct((NU, D), _f32),
            jax.ShapeDtypeStruct((NI, D), _f32),
            jax.ShapeDtypeStruct((NI, D), _f32),
        ],
    )(t2, xu_h, xu_c, xi_h, xi_c, m_uh, m_uc, m_ih, m_ic, degs,
      eu, ei, scales, shifts,
      Wu_eh, Wu_ec, Wi_eh, Wi_ec, Wz_u, Wh_u, Wc_u, Wz_i, Wh_i, Wc_i)


def _loss_body(zup, zip_, zun, zin, Wpu, Wpi, o):
    pu = jnp.dot(zup[...], Wpu[...], preferred_element_type=_f32)
    pi = jnp.dot(zip_[...], Wpi[...], preferred_element_type=_f32)
    cols = [jnp.sum(pu * pi, 1, keepdims=True)]
    for k in range(K):
        pik = jnp.dot(zin[:, pl.ds(k * 2 * D, 2 * D)], Wpi[...],
                      preferred_element_type=_f32)
        cols.append(jnp.sum(pu * pik, 1, keepdims=True))
    for k in range(K):
        puk = jnp.dot(zun[:, pl.ds(k * 2 * D, 2 * D)], Wpu[...],
                      preferred_element_type=_f32)
        cols.append(jnp.sum(puk * pi, 1, keepdims=True))
    scores = jnp.concatenate(cols, axis=1)
    m = jnp.max(scores, 1, keepdims=True)
    lse = jnp.log(jnp.sum(jnp.exp(scores - m), 1, keepdims=True)) + m
    loss = jnp.mean(lse[:, 0] - scores[:, 0])
    o[...] = jnp.broadcast_to(loss, (8, 128))


def _tc_loss(zu_pos, zi_pos, zu_neg2, zi_neg2, Wp_u, Wp_i):
    full = lambda s: pl.BlockSpec(s, lambda: (0, 0))
    return pl.pallas_call(
        _loss_body,
        in_specs=[full((B, 2 * D)), full((B, 2 * D)),
                  full((B, 2 * D * K)), full((B, 2 * D * K)),
                  full((2 * D, D)), full((2 * D, D))],
        out_specs=full((8, 128)),
        out_shape=jax.ShapeDtypeStruct((8, 128), _f32),
    )(zu_pos, zi_pos, zu_neg2, zi_neg2, Wp_u, Wp_i)


def _final_body(xu_th, xi_th, xu_tc, xi_tc, mu_h, mi_h, mu_c, mi_c,
                Wu_uh, Wi_uh, Wu_uc, Wi_uc, ou_h, oi_h, ou_c, oi_c):
    for x, m, w, o in ((xu_th, mu_h, Wu_uh, ou_h), (xi_th, mi_h, Wi_uh, oi_h),
                       (xu_tc, mu_c, Wu_uc, ou_c), (xi_tc, mi_c, Wi_uc, oi_c)):
        o[...] = x[...] + jnp.tanh(
            jnp.dot(m[...], w[...], preferred_element_type=_f32))


def _tc_final(xu_th, xi_th, xu_tc, xi_tc, mu_h, mi_h, mu_c, mi_c,
              Wu_uh, Wi_uh, Wu_uc, Wi_uc):
    spec = pl.BlockSpec((_RB, D), lambda i: (i, 0))
    w1 = pl.BlockSpec((D, D), lambda i: (0, 0))
    return pl.pallas_call(
        _final_body,
        grid=(_NB,),
        in_specs=[spec] * 8 + [w1] * 4,
        out_specs=[spec] * 4,
        out_shape=[jax.ShapeDtypeStruct((NU, D), _f32)] * 4,
    )(xu_th, xi_th, xu_tc, xi_tc, mu_h, mi_h, mu_c, mi_c,
      Wu_uh, Wi_uh, Wu_uc, Wi_uc)


def kernel(t_diff, adj_his, adj_ctx, adj_tgt_i2u, adj_tgt_u2i, tgt_u, tgt_i,
           tgt_u_neg, tgt_i_neg, xu_in_his, xi_in_his, xu_in_ctx, xi_in_ctx,
           embeds_u, embeds_i, g_uh, b_uh, g_ih, b_ih, g_uc, b_uc, g_ic, b_ic,
           Wu_eh, Wi_eh, Wu_ec, Wi_ec, Wu_uh, Wi_uh, Wu_uc, Wi_uc,
           Wz_u, Wh_u, Wc_u, Wz_i, Wh_i, Wc_i, Wp_u, Wp_i):
    stats = _tc_stats(xu_in_his, xi_in_his, xu_in_ctx, xi_in_ctx)
    sums = stats[0::2, :] / NU
    sqs = stats[1::2, :] / NU
    var = sqs - sums * sums
    g = jnp.stack([g_uh, g_ih, g_uc, g_ic])
    bb = jnp.stack([b_uh, b_ih, b_uc, b_ic])
    scales = g / jnp.sqrt(var + 1e-5)
    shifts = bb - sums * scales

    pad = E_PAD - E
    padsrc = (jnp.arange(pad, dtype=_i32) * 61) % NU
    padbad = NU + (jnp.arange(pad, dtype=_i32) % (ACC2 - NU))

    def prep(row):
        src = jnp.concatenate([row, padsrc]).reshape(-1, 128)
        dst = jnp.concatenate([row, padbad]).reshape(-1, 128)
        return src, dst

    s_h_s, s_h_d = prep(adj_his[0])
    d_h_s, d_h_d = prep(adj_his[1])
    s_c_s, s_c_d = prep(adj_ctx[0])
    d_c_s, d_c_d = prep(adj_ctx[1])
    zrows = jnp.zeros((STRIPE, D), _f32)
    zrows32 = jnp.zeros((WSTRIPE, HD), _f32)
    zdeg = jnp.zeros((WSTRIPE,), _f32)

    (m_uh, m_ih, m_uc, m_ic, dg_uh, dg_ih, dg_uc, dg_ic,
     _t0, _t1, _t2, _t3) = _sc_msgs(
        s_h_s, s_h_d, d_h_s, d_h_d, s_c_s, s_c_d, d_c_s, d_c_d,
        xi_in_his, xu_in_his, xi_in_ctx, xu_in_ctx, zrows32, zdeg)
    degs = jnp.stack([dg_uh, dg_ih, dg_uc, dg_ic], axis=1)

    t2 = t_diff.reshape(1, 1)
    zu_enc, zi_enc, xu_th, xu_tc, xi_th, xi_tc = _tc_main(
        t2, xu_in_his, xu_in_ctx, xi_in_his, xi_in_ctx,
        m_uh, m_uc, m_ih, m_ic, degs,
        embeds_u, embeds_i, scales, shifts,
        Wu_eh, Wu_ec, Wi_eh, Wi_ec, Wz_u, Wh_u, Wc_u, Wz_i, Wh_i, Wc_i)

    zu_pos, zi_pos, zu_neg, zi_neg, mu_h, mi_h, mu_c, mi_c = _sc_targets(
        zu_enc, zi_enc, xu_th, xi_th, xu_tc, xi_tc,
        tgt_u.reshape(-1, 128), tgt_i.reshape(-1, 128),
        tgt_u_neg.reshape(-1, 128), tgt_i_neg.reshape(-1, 128),
        adj_tgt_i2u[0].reshape(-1, 128), adj_tgt_i2u[1].reshape(-1, 128),
        adj_tgt_u2i[0].reshape(-1, 128), adj_tgt_u2i[1].reshape(-1, 128),
        zrows)

    lossmat = _tc_loss(zu_pos, zi_pos,
                       zu_neg.reshape(B, 2 * D * K),
                       zi_neg.reshape(B, 2 * D * K), Wp_u, Wp_i)
    loss = lossmat[0, 0]

    ou_h, oi_h, ou_c, oi_c = _tc_final(
        xu_th, xi_th, xu_tc, xi_tc, mu_h, mi_h, mu_c, mi_c,
        Wu_uh, Wi_uh, Wu_uc, Wi_uc)

    return (loss, zu_pos, zi_enc, ou_h, oi_h, ou_c, oi_c)


# speedup vs baseline: 7.2503x; 1.1053x over previous
import functools

import jax
import jax.numpy as jnp
from jax import lax
from jax.experimental import pallas as pl
from jax.experimental.pallas import tpu as pltpu
from jax.experimental.pallas import tpu_sc as plsc

NU = 50000
NI = 50000
D = 64
E = 800000
B = 4096
K = 5

NC = 2
NS = 16
HALF = NU // NC
ACC_ROWS = 25088
STRIPE = 1568
LAST_STRIPE = HALF - (NS - 1) * STRIPE
E_PAD = 802816
NCHUNK = 49

_f32 = jnp.float32
_i32 = jnp.int32


HD = D // 2
ACC2 = 50048
WSTRIPE = 3128
WLAST = NU - (NS - 1) * WSTRIPE


def _sc_edge_phase(srcr, dstr, tab, outr, degr, do_deg, zrows, zdeg,
                   src_v, dst_v, src_v1, dst_v1, rows, ones_v, acc, dacc,
                   gsem, ssem, isem, cid, sid):
    pltpu.sync_copy(zrows, acc.at[pl.ds(sid * WSTRIPE, WSTRIPE)])
    pltpu.sync_copy(zdeg, dacc.at[pl.ds(sid * WSTRIPE, WSTRIPE)])
    plsc.subcore_barrier()

    for cc in range(NC):
        @pl.when(cid == cc)
        def _(cc=cc):
            tabh = tab.at[cc]
            my_deg = do_deg == cc

            def proc(sv, dv, i):
                gd = [None] * 8
                sd = [None] * 8
                gd[0] = pltpu.async_copy(tabh.at[sv.at[0]],
                                         rows.at[pl.ds(0, 128)], gsem)
                for j in range(8):
                    if j + 1 < 8:
                        if j >= 4:
                            sd[j - 4].wait()
                        gd[j + 1] = pltpu.async_copy(
                            tabh.at[sv.at[j + 1]],
                            rows.at[pl.ds(((j + 1) % 5) * 128, 128)], gsem)
                    gd[j].wait()
                    sd[j] = pltpu.async_copy(
                        rows.at[pl.ds((j % 5) * 128, 128)],
                        acc.at[dv.at[j]], ssem, add=True)
                    if my_deg:
                        pltpu.sync_copy(ones_v, dacc.at[dv.at[j]],
                                        add=True)
                for j in range(3, 8):
                    sd[j].wait()

            def erow(i):
                return sid * (NCHUNK * 8) + i * 8

            pltpu.sync_copy(srcr.at[pl.ds(erow(0), 8)], src_v)
            pltpu.sync_copy(dstr.at[pl.ds(erow(0), 8)], dst_v)

            def pair(p, carry):
                i0 = 2 * p
                dA = pltpu.async_copy(srcr.at[pl.ds(erow(i0 + 1), 8)],
                                      src_v1, isem)
                dB = pltpu.async_copy(dstr.at[pl.ds(erow(i0 + 1), 8)],
                                      dst_v1, isem)
                proc(src_v, dst_v, i0)
                dA.wait()
                dB.wait()
                dC = pltpu.async_copy(srcr.at[pl.ds(erow(i0 + 2), 8)],
                                      src_v, isem)
                dD = pltpu.async_copy(dstr.at[pl.ds(erow(i0 + 2), 8)],
                                      dst_v, isem)
                proc(src_v1, dst_v1, i0 + 1)
                dC.wait()
                dD.wait()
                return carry

            lax.fori_loop(0, (NCHUNK - 1) // 2, pair, 0)
            proc(src_v, dst_v, NCHUNK - 1)

    plsc.subcore_barrier()

    nrows = [WSTRIPE, WLAST]
    for cc in range(NC):
        for last in range(2):
            sel = (sid == NS - 1) if last else (sid < NS - 1)
            r0c = (NS - 1) * WSTRIPE if last else None

            @pl.when((cid == cc) & sel)
            def _(cc=cc, n=nrows[last], last=last):
                r0 = (NS - 1) * WSTRIPE if last else sid * WSTRIPE
                pltpu.sync_copy(acc.at[pl.ds(r0, n)],
                                outr.at[pl.ds(r0, n), pl.ds(cc * HD, HD)])

            if do_deg == cc:
                @pl.when((cid == cc) & sel)
                def _(n=nrows[last], last=last):
                    r0 = (NS - 1) * WSTRIPE if last else sid * WSTRIPE
                    pltpu.sync_copy(dacc.at[pl.ds(r0, n)],
                                    degr.at[pl.ds(r0, n)])


def _sc_msgs_body(s_h_s, s_h_d, d_h_s, d_h_d, s_c_s, s_c_d, d_c_s, d_c_d,
                  r_ih, r_uh, r_ic, r_uc, zrows, zdeg,
                  o_uh, o_ih, o_uc, o_ic, dg_uh, dg_ih, dg_uc, dg_ic,
                  tb_ih, tb_uh, tb_ic, tb_uc,
                  src_v, dst_v, src_v1, dst_v1, rows, ones_v, acc, dacc,
                  gsem, ssem, isem):
    cid = lax.axis_index("c")
    sid = lax.axis_index("s")
    for t in range(8):
        ones_v[pl.ds(t * 16, 16)] = jnp.full((16,), 1.0, _f32)

    for raw, tb in ((r_ih, tb_ih), (r_uh, tb_uh), (r_ic, tb_ic),
                    (r_uc, tb_uc)):
        for cc in range(NC):
            @pl.when(cid == cc)
            def _(raw=raw, tb=tb, cc=cc):
                base = sid * WSTRIPE

                def cp(r0, n):
                    pltpu.sync_copy(
                        raw.at[pl.ds(r0, n), pl.ds(cc * HD, HD)],
                        rows.at[pl.ds(0, n)])
                    pltpu.sync_copy(rows.at[pl.ds(0, n)],
                                    tb.at[cc].at[pl.ds(r0, n)])

                def q4(q, carry):
                    cp(base + q * 640, 640)
                    return carry

                lax.fori_loop(0, 4, q4, 0)

                @pl.when(sid < NS - 1)
                def _():
                    cp(base + 2560, WSTRIPE - 2560)

                @pl.when(sid == NS - 1)
                def _():
                    cp(base + 2560, WLAST - 2560)
    plsc.subcore_barrier()

    phases = (
        (d_h_s, s_h_d, tb_ih, o_uh, dg_uh, 0),
        (s_h_s, d_h_d, tb_uh, o_ih, dg_ih, 1),
        (d_c_s, s_c_d, tb_ic, o_uc, dg_uc, 0),
        (s_c_s, d_c_d, tb_uc, o_ic, dg_ic, 1),
    )
    for srcr, dstr, tab, outr, degr, do_deg in phases:
        _sc_edge_phase(srcr, dstr, tab, outr, degr, do_deg, zrows, zdeg,
                       src_v, dst_v, src_v1, dst_v1, rows, ones_v, acc,
                       dacc, gsem, ssem, isem, cid, sid)


_sc_msgs = functools.partial(
    pl.kernel,
    out_type=tuple(
        [jax.ShapeDtypeStruct((NU, D), _f32) for _ in range(4)]
        + [jax.ShapeDtypeStruct((NU,), _f32) for _ in range(4)]
        + [jax.ShapeDtypeStruct((NC, NU, HD), _f32) for _ in range(4)]),
    mesh=plsc.VectorSubcoreMesh(core_axis_name="c", subcore_axis_name="s"),
    scratch_types=[
        pltpu.VMEM((8, 128), _i32),
        pltpu.VMEM((8, 128), _i32),
        pltpu.VMEM((8, 128), _i32),
        pltpu.VMEM((8, 128), _i32),
        pltpu.VMEM((640, HD), _f32),
        pltpu.VMEM((128,), _f32),
        pltpu.VMEM_SHARED((ACC2, HD), _f32),
        pltpu.VMEM_SHARED((ACC2,), _f32),
        pltpu.SemaphoreType.DMA,
        pltpu.SemaphoreType.DMA,
        pltpu.SemaphoreType.DMA,
    ],
    compiler_params=pltpu.CompilerParams(use_tc_tiling_on_sc=False),
)(_sc_msgs_body)


def _sc_targets_body(zu_enc, zi_enc, xu_th, xi_th, xu_tc, xi_tc,
                     tgtu, tgti, tgtun, tgtin,
                     i2u_s, i2u_d, u2i_s, u2i_d, zrows,
                     zu_pos, zi_pos, zu_neg, zi_neg,
                     mu_h, mi_h, mu_c, mi_c,
                     idx_v, rows128, rows64, ldst, acc, gsem, ssem):
    cid = lax.axis_index("c")
    sid = lax.axis_index("s")
    wid = sid * NC + cid

    for idx2d, tab, outr, n_per_w in (
            (tgtu, zu_enc, zu_pos, 1),
            (tgti, zi_enc, zi_pos, 1),
            (tgtun, zu_enc, zu_neg, 5),
            (tgtin, zi_enc, zi_neg, 5),
    ):
        for q in range(n_per_w):
            crow = wid * n_per_w + q
            pltpu.sync_copy(idx2d.at[pl.ds(crow, 1)], idx_v.at[pl.ds(0, 1)])
            pltpu.async_copy(tab.at[idx_v.at[0]], rows128, gsem).wait()
            pltpu.sync_copy(rows128, outr.at[pl.ds(crow * 128, 128)])

    base_row = cid * HALF
    trash = HALF + 8 + sid * 4
    for src2d, dst2d, tab, outr in (
            (i2u_s, i2u_d, xi_th, mu_h),
            (u2i_s, u2i_d, xu_th, mi_h),
            (i2u_s, i2u_d, xi_tc, mu_c),
            (u2i_s, u2i_d, xu_tc, mi_c),
    ):
        pltpu.sync_copy(zrows, acc.at[pl.ds(sid * STRIPE, STRIPE)])
        plsc.subcore_barrier()
        for q in range(2):
            crow = sid * 2 + q
            pltpu.sync_copy(src2d.at[pl.ds(crow, 1)], idx_v.at[pl.ds(0, 1)])
            pltpu.sync_copy(dst2d.at[pl.ds(crow, 1)], idx_v.at[pl.ds(1, 1)])
            dsc = pltpu.async_copy(tab.at[idx_v.at[0]], rows64, gsem)
            for t in range(8):
                v = idx_v[1, pl.ds(t * 16, 16)]
                tl = v - base_row
                m = (tl >= 0) & (tl < HALF)
                ldst[0, pl.ds(t * 16, 16)] = jnp.where(m, tl, trash)
            dsc.wait()
            pltpu.async_copy(rows64, acc.at[ldst.at[0]], ssem,
                             add=True).wait()
        plsc.subcore_barrier()

        @pl.when(sid < NS - 1)
        def _():
            pltpu.sync_copy(acc.at[pl.ds(sid * STRIPE, STRIPE)],
                            outr.at[pl.ds(base_row + sid * STRIPE, STRIPE)])

        @pl.when(sid == NS - 1)
        def _():
            pltpu.sync_copy(acc.at[pl.ds((NS - 1) * STRIPE, LAST_STRIPE)],
                            outr.at[pl.ds(base_row + (NS - 1) * STRIPE,
                                          LAST_STRIPE)])


_sc_targets = functools.partial(
    pl.kernel,
    out_type=(
        jax.ShapeDtypeStruct((B, 2 * D), _f32),
        jax.ShapeDtypeStruct((B, 2 * D), _f32),
        jax.ShapeDtypeStruct((B * K, 2 * D), _f32),
        jax.ShapeDtypeStruct((B * K, 2 * D), _f32),
        jax.ShapeDtypeStruct((NU, D), _f32),
        jax.ShapeDtypeStruct((NI, D), _f32),
        jax.ShapeDtypeStruct((NU, D), _f32),
        jax.ShapeDtypeStruct((NI, D), _f32),
    ),
    mesh=plsc.VectorSubcoreMesh(core_axis_name="c", subcore_axis_name="s"),
    scratch_types=[
        pltpu.VMEM((8, 128), _i32),
        pltpu.VMEM((128, 2 * D), _f32),
        pltpu.VMEM((128, D), _f32),
        pltpu.VMEM((8, 128), _i32),
        pltpu.VMEM_SHARED((ACC_ROWS, D), _f32),
        pltpu.SemaphoreType.DMA,
        pltpu.SemaphoreType.DMA,
    ],
    compiler_params=pltpu.CompilerParams(use_tc_tiling_on_sc=False),
)(_sc_targets_body)


_RB = 1000
_NB = NU // _RB


def _stats_body(a, b, c, d, o):
    i = pl.program_id(0)

    @pl.when(i == 0)
    def _():
        o[...] = jnp.zeros_like(o)

    for r, x in enumerate((a, b, c, d)):
        xv = x[...]
        o[pl.ds(2 * r, 1), :] += jnp.sum(xv, 0, keepdims=True)
        o[pl.ds(2 * r + 1, 1), :] += jnp.sum(xv * xv, 0, keepdims=True)


def _tc_stats(xuh, xih, xuc, xic):
    spec = pl.BlockSpec((_RB, D), lambda i: (i, 0))
    return pl.pallas_call(
        _stats_body,
        grid=(_NB,),
        in_specs=[spec] * 4,
        out_specs=pl.BlockSpec((8, D), lambda i: (0, 0)),
        out_shape=jax.ShapeDtypeStruct((8, D), _f32),
    )(xuh, xih, xuc, xic)


def _main_body(t, xu_h, xu_c, xi_h, xi_c, m_uh, m_uc, m_ih, m_ic, degs,
               eu, ei, scv, shv,
               Wu_eh, Wu_ec, Wi_eh, Wi_ec, Wz_u, Wh_u, Wc_u, Wz_i, Wh_i, Wc_i,
               zu_enc, zi_enc, xu_th, xu_tc, xi_th, xi_tc):
    decay = jnp.exp(-t[0, 0])
    sc = scv[...]
    sh = shv[...]
    dg = degs[...]

    def bn(x, r):
        return x[...] * sc[r:r + 1, :] + sh[r:r + 1, :]

    def evo(xraw, rx, m, rg, p, W, e):
        h = (decay * bn(xraw, rx) + m[...] * sc[rg:rg + 1, :]
             + dg[:, p:p + 1] * sh[rg:rg + 1, :])
        return jnp.tanh(jnp.dot(h, W[...], preferred_element_type=_f32)) + e

    def side(xrh, xrc, rxh, rxc, mh, mc, rgh, rgc, ph, pc, e,
             Weh, Wec, Wz, Wh, Wc, zenc, xth, xtc):
        ev = e[...]
        xmh = evo(xrh, rxh, mh, rgh, ph, Weh, ev)
        xmc = evo(xrc, rxc, mc, rgc, pc, Wec, ev)

        def two(Wr):
            return (jnp.dot(xmh, Wr[pl.ds(0, D), :],
                            preferred_element_type=_f32) +
                    jnp.dot(xmc, Wr[pl.ds(D, D), :],
                            preferred_element_type=_f32))

        zenc[:, pl.ds(0, D)] = jnp.tanh(two(Wz))
        zenc[:, pl.ds(D, D)] = ev
        xth[...] = jnp.tanh(two(Wh))
        xtc[...] = jnp.tanh(two(Wc))

    side(xu_h, xu_c, 0, 2, m_uh, m_uc, 1, 3, 0, 2, eu,
         Wu_eh, Wu_ec, Wz_u, Wh_u, Wc_u, zu_enc, xu_th, xu_tc)
    side(xi_h, xi_c, 1, 3, m_ih, m_ic, 0, 2, 1, 3, ei,
         Wi_eh, Wi_ec, Wz_i, Wh_i, Wc_i, zi_enc, xi_th, xi_tc)


def _tc_main(t2, xu_h, xu_c, xi_h, xi_c, m_uh, m_uc, m_ih, m_ic, degs,
             eu, ei, scales, shifts,
             Wu_eh, Wu_ec, Wi_eh, Wi_ec, Wz_u, Wh_u, Wc_u, Wz_i, Wh_i, Wc_i):
    spec = pl.BlockSpec((_RB, D), lambda i: (i, 0))
    spec2 = pl.BlockSpec((_RB, 2 * D), lambda i: (i, 0))
    dspec = pl.BlockSpec((_RB, 4), lambda i: (i, 0))
    w1 = pl.BlockSpec((D, D), lambda i: (0, 0))
    w2 = pl.BlockSpec((2 * D, D), lambda i: (0, 0))
    cspec = pl.BlockSpec((4, D), lambda i: (0, 0))
    tspec = pl.BlockSpec((1, 1), lambda i: (0, 0))
    return pl.pallas_call(
        _main_body,
        grid=(_NB,),
        in_specs=[tspec] + [spec] * 8 + [dspec] + [spec] * 2 + [cspec] * 2
                 + [w1] * 4 + [w2] * 6,
        out_specs=[spec2, spec2, spec, spec, spec, spec],
        out_shape=[
            jax.ShapeDtypeStruct((NU, 2 * D), _f32),
            jax.ShapeDtypeStruct((NI, 2 * D), _f32),
            jax.ShapeDtypeStruct((NU, D), _f32),
            jax.ShapeDtypeStruct((NU, D), _f32),
            jax.ShapeDtypeStruct((NI, D), _f32),
            jax.ShapeDtypeStruct((NI, D), _f32),
        ],
    )(t2, xu_h, xu_c, xi_h, xi_c, m_uh, m_uc, m_ih, m_ic, degs,
      eu, ei, scales, shifts,
      Wu_eh, Wu_ec, Wi_eh, Wi_ec, Wz_u, Wh_u, Wc_u, Wz_i, Wh_i, Wc_i)


def _loss_body(zup, zip_, zun, zin, Wpu, Wpi, o):
    pu = jnp.dot(zup[...], Wpu[...], preferred_element_type=_f32)
    pi = jnp.dot(zip_[...], Wpi[...], preferred_element_type=_f32)
    cols = [jnp.sum(pu * pi, 1, keepdims=True)]
    for k in range(K):
        pik = jnp.dot(zin[:, pl.ds(k * 2 * D, 2 * D)], Wpi[...],
                      preferred_element_type=_f32)
        cols.append(jnp.sum(pu * pik, 1, keepdims=True))
    for k in range(K):
        puk = jnp.dot(zun[:, pl.ds(k * 2 * D, 2 * D)], Wpu[...],
                      preferred_element_type=_f32)
        cols.append(jnp.sum(puk * pi, 1, keepdims=True))
    scores = jnp.concatenate(cols, axis=1)
    m = jnp.max(scores, 1, keepdims=True)
    lse = jnp.log(jnp.sum(jnp.exp(scores - m), 1, keepdims=True)) + m
    loss = jnp.mean(lse[:, 0] - scores[:, 0])
    o[...] = jnp.broadcast_to(loss, (8, 128))


def _tc_loss(zu_pos, zi_pos, zu_neg2, zi_neg2, Wp_u, Wp_i):
    full = lambda s: pl.BlockSpec(s, lambda: (0, 0))
    return pl.pallas_call(
        _loss_body,
        in_specs=[full((B, 2 * D)), full((B, 2 * D)),
                  full((B, 2 * D * K)), full((B, 2 * D * K)),
                  full((2 * D, D)), full((2 * D, D))],
        out_specs=full((8, 128)),
        out_shape=jax.ShapeDtypeStruct((8, 128), _f32),
    )(zu_pos, zi_pos, zu_neg2, zi_neg2, Wp_u, Wp_i)


def _final_body(xu_th, xi_th, xu_tc, xi_tc, mu_h, mi_h, mu_c, mi_c,
                Wu_uh, Wi_uh, Wu_uc, Wi_uc, ou_h, oi_h, ou_c, oi_c):
    for x, m, w, o in ((xu_th, mu_h, Wu_uh, ou_h), (xi_th, mi_h, Wi_uh, oi_h),
                       (xu_tc, mu_c, Wu_uc, ou_c), (xi_tc, mi_c, Wi_uc, oi_c)):
        o[...] = x[...] + jnp.tanh(
            jnp.dot(m[...], w[...], preferred_element_type=_f32))


def _tc_final(xu_th, xi_th, xu_tc, xi_tc, mu_h, mi_h, mu_c, mi_c,
              Wu_uh, Wi_uh, Wu_uc, Wi_uc):
    spec = pl.BlockSpec((_RB, D), lambda i: (i, 0))
    w1 = pl.BlockSpec((D, D), lambda i: (0, 0))
    return pl.pallas_call(
        _final_body,
        grid=(_NB,),
        in_specs=[spec] * 8 + [w1] * 4,
        out_specs=[spec] * 4,
        out_shape=[jax.ShapeDtypeStruct((NU, D), _f32)] * 4,
    )(xu_th, xi_th, xu_tc, xi_tc, mu_h, mi_h, mu_c, mi_c,
      Wu_uh, Wi_uh, Wu_uc, Wi_uc)


def kernel(t_diff, adj_his, adj_ctx, adj_tgt_i2u, adj_tgt_u2i, tgt_u, tgt_i,
           tgt_u_neg, tgt_i_neg, xu_in_his, xi_in_his, xu_in_ctx, xi_in_ctx,
           embeds_u, embeds_i, g_uh, b_uh, g_ih, b_ih, g_uc, b_uc, g_ic, b_ic,
           Wu_eh, Wi_eh, Wu_ec, Wi_ec, Wu_uh, Wi_uh, Wu_uc, Wi_uc,
           Wz_u, Wh_u, Wc_u, Wz_i, Wh_i, Wc_i, Wp_u, Wp_i):
    stats = _tc_stats(xu_in_his, xi_in_his, xu_in_ctx, xi_in_ctx)
    sums = stats[0::2, :] / NU
    sqs = stats[1::2, :] / NU
    var = sqs - sums * sums
    g = jnp.stack([g_uh, g_ih, g_uc, g_ic])
    bb = jnp.stack([b_uh, b_ih, b_uc, b_ic])
    scales = g / jnp.sqrt(var + 1e-5)
    shifts = bb - sums * scales

    pad = E_PAD - E
    padsrc = (jnp.arange(pad, dtype=_i32) * 61) % NU
    padbad = NU + (jnp.arange(pad, dtype=_i32) % (ACC2 - NU))

    def prep(row):
        src = jnp.concatenate([row, padsrc]).reshape(-1, 128)
        dst = jnp.concatenate([row, padbad]).reshape(-1, 128)
        return src, dst

    s_h_s, s_h_d = prep(adj_his[0])
    d_h_s, d_h_d = prep(adj_his[1])
    s_c_s, s_c_d = prep(adj_ctx[0])
    d_c_s, d_c_d = prep(adj_ctx[1])
    zrows = jnp.zeros((STRIPE, D), _f32)
    zrows32 = jnp.zeros((WSTRIPE, HD), _f32)
    zdeg = jnp.zeros((WSTRIPE,), _f32)

    (m_uh, m_ih, m_uc, m_ic, dg_uh, dg_ih, dg_uc, dg_ic,
     _t0, _t1, _t2, _t3) = _sc_msgs(
        s_h_s, s_h_d, d_h_s, d_h_d, s_c_s, s_c_d, d_c_s, d_c_d,
        xi_in_his, xu_in_his, xi_in_ctx, xu_in_ctx, zrows32, zdeg)
    degs = jnp.stack([dg_uh, dg_ih, dg_uc, dg_ic], axis=1)

    t2 = t_diff.reshape(1, 1)
    zu_enc, zi_enc, xu_th, xu_tc, xi_th, xi_tc = _tc_main(
        t2, xu_in_his, xu_in_ctx, xi_in_his, xi_in_ctx,
        m_uh, m_uc, m_ih, m_ic, degs,
        embeds_u, embeds_i, scales, shifts,
        Wu_eh, Wu_ec, Wi_eh, Wi_ec, Wz_u, Wh_u, Wc_u, Wz_i, Wh_i, Wc_i)

    zu_pos, zi_pos, zu_neg, zi_neg, mu_h, mi_h, mu_c, mi_c = _sc_targets(
        zu_enc, zi_enc, xu_th, xi_th, xu_tc, xi_tc,
        tgt_u.reshape(-1, 128), tgt_i.reshape(-1, 128),
        tgt_u_neg.reshape(-1, 128), tgt_i_neg.reshape(-1, 128),
        adj_tgt_i2u[0].reshape(-1, 128), adj_tgt_i2u[1].reshape(-1, 128),
        adj_tgt_u2i[0].reshape(-1, 128), adj_tgt_u2i[1].reshape(-1, 128),
        zrows)

    lossmat = _tc_loss(zu_pos, zi_pos,
                       zu_neg.reshape(B, 2 * D * K),
                       zi_neg.reshape(B, 2 * D * K), Wp_u, Wp_i)
    loss = lossmat[0, 0]

    ou_h, oi_h, ou_c, oi_c = _tc_final(
        xu_th, xi_th, xu_tc, xi_tc, mu_h, mi_h, mu_c, mi_c,
        Wu_uh, Wi_uh, Wu_uc, Wi_uc)

    return (loss, zu_pos, zi_enc, ou_h, oi_h, ou_c, oi_c)


# speedup vs baseline: 7.3744x; 1.0171x over previous
import functools

import jax
import jax.numpy as jnp
from jax import lax
from jax.experimental import pallas as pl
from jax.experimental.pallas import tpu as pltpu
from jax.experimental.pallas import tpu_sc as plsc

NU = 50000
NI = 50000
D = 64
E = 800000
B = 4096
K = 5

NC = 2
NS = 16
HALF = NU // NC
ACC_ROWS = 25088
STRIPE = 1568
LAST_STRIPE = HALF - (NS - 1) * STRIPE
E_PAD = 802816
NCHUNK = 49

_f32 = jnp.float32
_i32 = jnp.int32


HD = D // 2
ACC2 = 50048
WSTRIPE = 3128
WLAST = NU - (NS - 1) * WSTRIPE


def _sc_edge_phase(srcr, dstr, tab, outr, degr, do_deg, zrows, zdeg,
                   src_v, dst_v, src_v1, dst_v1, rows, ones_v, acc, dacc,
                   gsem, ssem, isem, dsem, cid, sid):
    pltpu.sync_copy(zrows, acc.at[pl.ds(sid * WSTRIPE, WSTRIPE)])
    pltpu.sync_copy(zdeg, dacc.at[pl.ds(sid * WSTRIPE, WSTRIPE)])
    plsc.subcore_barrier()

    for cc in range(NC):
        @pl.when(cid == cc)
        def _(cc=cc):
            tabh = tab.at[cc]
            my_deg = do_deg == cc

            def proc(sv, dv, i):
                gd = [None] * 8
                sd = [None] * 8
                dd = [None] * 8
                gd[0] = pltpu.async_copy(tabh.at[sv.at[0]],
                                         rows.at[pl.ds(0, 128)], gsem)
                for j in range(8):
                    if j + 1 < 8:
                        if j >= 4:
                            sd[j - 4].wait()
                        gd[j + 1] = pltpu.async_copy(
                            tabh.at[sv.at[j + 1]],
                            rows.at[pl.ds(((j + 1) % 5) * 128, 128)], gsem)
                    gd[j].wait()
                    sd[j] = pltpu.async_copy(
                        rows.at[pl.ds((j % 5) * 128, 128)],
                        acc.at[dv.at[j]], ssem, add=True)
                    if my_deg:
                        dd[j] = pltpu.async_copy(
                            ones_v, dacc.at[dv.at[j]], dsem, add=True)
                for j in range(3, 8):
                    sd[j].wait()
                if my_deg:
                    for j in range(8):
                        dd[j].wait()

            def erow(i):
                return sid * (NCHUNK * 8) + i * 8

            pltpu.sync_copy(srcr.at[pl.ds(erow(0), 8)], src_v)
            pltpu.sync_copy(dstr.at[pl.ds(erow(0), 8)], dst_v)

            def pair(p, carry):
                i0 = 2 * p
                dA = pltpu.async_copy(srcr.at[pl.ds(erow(i0 + 1), 8)],
                                      src_v1, isem)
                dB = pltpu.async_copy(dstr.at[pl.ds(erow(i0 + 1), 8)],
                                      dst_v1, isem)
                proc(src_v, dst_v, i0)
                dA.wait()
                dB.wait()
                dC = pltpu.async_copy(srcr.at[pl.ds(erow(i0 + 2), 8)],
                                      src_v, isem)
                dD = pltpu.async_copy(dstr.at[pl.ds(erow(i0 + 2), 8)],
                                      dst_v, isem)
                proc(src_v1, dst_v1, i0 + 1)
                dC.wait()
                dD.wait()
                return carry

            lax.fori_loop(0, (NCHUNK - 1) // 2, pair, 0)
            proc(src_v, dst_v, NCHUNK - 1)

    plsc.subcore_barrier()

    nrows = [WSTRIPE, WLAST]
    for cc in range(NC):
        for last in range(2):
            sel = (sid == NS - 1) if last else (sid < NS - 1)
            r0c = (NS - 1) * WSTRIPE if last else None

            @pl.when((cid == cc) & sel)
            def _(cc=cc, n=nrows[last], last=last):
                r0 = (NS - 1) * WSTRIPE if last else sid * WSTRIPE
                pltpu.sync_copy(acc.at[pl.ds(r0, n)],
                                outr.at[pl.ds(r0, n), pl.ds(cc * HD, HD)])

            if do_deg == cc:
                @pl.when((cid == cc) & sel)
                def _(n=nrows[last], last=last):
                    r0 = (NS - 1) * WSTRIPE if last else sid * WSTRIPE
                    pltpu.sync_copy(dacc.at[pl.ds(r0, n)],
                                    degr.at[pl.ds(r0, n)])


def _sc_msgs_body(s_h_s, s_h_d, d_h_s, d_h_d, s_c_s, s_c_d, d_c_s, d_c_d,
                  r_ih, r_uh, r_ic, r_uc, zrows, zdeg,
                  o_uh, o_ih, o_uc, o_ic, dg_uh, dg_ih, dg_uc, dg_ic,
                  tb_ih, tb_uh, tb_ic, tb_uc,
                  src_v, dst_v, src_v1, dst_v1, rows, ones_v, acc, dacc,
                  gsem, ssem, isem, dsem):
    cid = lax.axis_index("c")
    sid = lax.axis_index("s")
    for t in range(8):
        ones_v[pl.ds(t * 16, 16)] = jnp.full((16,), 1.0, _f32)

    for raw, tb in ((r_ih, tb_ih), (r_uh, tb_uh), (r_ic, tb_ic),
                    (r_uc, tb_uc)):
        for cc in range(NC):
            @pl.when(cid == cc)
            def _(raw=raw, tb=tb, cc=cc):
                base = sid * WSTRIPE

                def cp(r0, n):
                    pltpu.sync_copy(
                        raw.at[pl.ds(r0, n), pl.ds(cc * HD, HD)],
                        rows.at[pl.ds(0, n)])
                    pltpu.sync_copy(rows.at[pl.ds(0, n)],
                                    tb.at[cc].at[pl.ds(r0, n)])

                def q4(q, carry):
                    cp(base + q * 640, 640)
                    return carry

                lax.fori_loop(0, 4, q4, 0)

                @pl.when(sid < NS - 1)
                def _():
                    cp(base + 2560, WSTRIPE - 2560)

                @pl.when(sid == NS - 1)
                def _():
                    cp(base + 2560, WLAST - 2560)
    plsc.subcore_barrier()

    phases = (
        (d_h_s, s_h_d, tb_ih, o_uh, dg_uh, 0),
        (s_h_s, d_h_d, tb_uh, o_ih, dg_ih, 1),
        (d_c_s, s_c_d, tb_ic, o_uc, dg_uc, 0),
        (s_c_s, d_c_d, tb_uc, o_ic, dg_ic, 1),
    )
    for srcr, dstr, tab, outr, degr, do_deg in phases:
        _sc_edge_phase(srcr, dstr, tab, outr, degr, do_deg, zrows, zdeg,
                       src_v, dst_v, src_v1, dst_v1, rows, ones_v, acc,
                       dacc, gsem, ssem, isem, dsem, cid, sid)


_sc_msgs = functools.partial(
    pl.kernel,
    out_type=tuple(
        [jax.ShapeDtypeStruct((NU, D), _f32) for _ in range(4)]
        + [jax.ShapeDtypeStruct((NU,), _f32) for _ in range(4)]
        + [jax.ShapeDtypeStruct((NC, NU, HD), _f32) for _ in range(4)]),
    mesh=plsc.VectorSubcoreMesh(core_axis_name="c", subcore_axis_name="s"),
    scratch_types=[
        pltpu.VMEM((8, 128), _i32),
        pltpu.VMEM((8, 128), _i32),
        pltpu.VMEM((8, 128), _i32),
        pltpu.VMEM((8, 128), _i32),
        pltpu.VMEM((640, HD), _f32),
        pltpu.VMEM((128,), _f32),
        pltpu.VMEM_SHARED((ACC2, HD), _f32),
        pltpu.VMEM_SHARED((ACC2,), _f32),
        pltpu.SemaphoreType.DMA,
        pltpu.SemaphoreType.DMA,
        pltpu.SemaphoreType.DMA,
        pltpu.SemaphoreType.DMA,
    ],
    compiler_params=pltpu.CompilerParams(use_tc_tiling_on_sc=False),
)(_sc_msgs_body)


def _sc_targets_body(zu_enc, zi_enc, xu_th, xi_th, xu_tc, xi_tc,
                     tgtu, tgti, tgtun, tgtin,
                     i2u_s, i2u_d, u2i_s, u2i_d, zrows,
                     zu_pos, zi_pos, zu_neg, zi_neg,
                     mu_h, mi_h, mu_c, mi_c,
                     idx_v, rows128, rows64, ldst, acc, gsem, ssem):
    cid = lax.axis_index("c")
    sid = lax.axis_index("s")
    wid = sid * NC + cid

    for idx2d, tab, outr, n_per_w in (
            (tgtu, zu_enc, zu_pos, 1),
            (tgti, zi_enc, zi_pos, 1),
            (tgtun, zu_enc, zu_neg, 5),
            (tgtin, zi_enc, zi_neg, 5),
    ):
        for q in range(n_per_w):
            crow = wid * n_per_w + q
            pltpu.sync_copy(idx2d.at[pl.ds(crow, 1)], idx_v.at[pl.ds(0, 1)])
            pltpu.async_copy(tab.at[idx_v.at[0]], rows128, gsem).wait()
            pltpu.sync_copy(rows128, outr.at[pl.ds(crow * 128, 128)])

    base_row = cid * HALF
    trash = HALF + 8 + sid * 4
    for src2d, dst2d, tab, outr in (
            (i2u_s, i2u_d, xi_th, mu_h),
            (u2i_s, u2i_d, xu_th, mi_h),
            (i2u_s, i2u_d, xi_tc, mu_c),
            (u2i_s, u2i_d, xu_tc, mi_c),
    ):
        pltpu.sync_copy(zrows, acc.at[pl.ds(sid * STRIPE, STRIPE)])
        plsc.subcore_barrier()
        for q in range(2):
            crow = sid * 2 + q
            pltpu.sync_copy(src2d.at[pl.ds(crow, 1)], idx_v.at[pl.ds(0, 1)])
            pltpu.sync_copy(dst2d.at[pl.ds(crow, 1)], idx_v.at[pl.ds(1, 1)])
            dsc = pltpu.async_copy(tab.at[idx_v.at[0]], rows64, gsem)
            for t in range(8):
                v = idx_v[1, pl.ds(t * 16, 16)]
                tl = v - base_row
                m = (tl >= 0) & (tl < HALF)
                ldst[0, pl.ds(t * 16, 16)] = jnp.where(m, tl, trash)
            dsc.wait()
            pltpu.async_copy(rows64, acc.at[ldst.at[0]], ssem,
                             add=True).wait()
        plsc.subcore_barrier()

        @pl.when(sid < NS - 1)
        def _():
            pltpu.sync_copy(acc.at[pl.ds(sid * STRIPE, STRIPE)],
                            outr.at[pl.ds(base_row + sid * STRIPE, STRIPE)])

        @pl.when(sid == NS - 1)
        def _():
            pltpu.sync_copy(acc.at[pl.ds((NS - 1) * STRIPE, LAST_STRIPE)],
                            outr.at[pl.ds(base_row + (NS - 1) * STRIPE,
                                          LAST_STRIPE)])


_sc_targets = functools.partial(
    pl.kernel,
    out_type=(
        jax.ShapeDtypeStruct((B, 2 * D), _f32),
        jax.ShapeDtypeStruct((B, 2 * D), _f32),
        jax.ShapeDtypeStruct((B * K, 2 * D), _f32),
        jax.ShapeDtypeStruct((B * K, 2 * D), _f32),
        jax.ShapeDtypeStruct((NU, D), _f32),
        jax.ShapeDtypeStruct((NI, D), _f32),
        jax.ShapeDtypeStruct((NU, D), _f32),
        jax.ShapeDtypeStruct((NI, D), _f32),
    ),
    mesh=plsc.VectorSubcoreMesh(core_axis_name="c", subcore_axis_name="s"),
    scratch_types=[
        pltpu.VMEM((8, 128), _i32),
        pltpu.VMEM((128, 2 * D), _f32),
        pltpu.VMEM((128, D), _f32),
        pltpu.VMEM((8, 128), _i32),
        pltpu.VMEM_SHARED((ACC_ROWS, D), _f32),
        pltpu.SemaphoreType.DMA,
        pltpu.SemaphoreType.DMA,
    ],
    compiler_params=pltpu.CompilerParams(use_tc_tiling_on_sc=False),
)(_sc_targets_body)


_RB = 1000
_NB = NU // _RB


def _stats_body(a, b, c, d, o):
    i = pl.program_id(0)

    @pl.when(i == 0)
    def _():
        o[...] = jnp.zeros_like(o)

    for r, x in enumerate((a, b, c, d)):
        xv = x[...]
        o[pl.ds(2 * r, 1), :] += jnp.sum(xv, 0, keepdims=True)
        o[pl.ds(2 * r + 1, 1), :] += jnp.sum(xv * xv, 0, keepdims=True)


def _tc_stats(xuh, xih, xuc, xic):
    spec = pl.BlockSpec((_RB, D), lambda i: (i, 0))
    return pl.pallas_call(
        _stats_body,
        grid=(_NB,),
        in_specs=[spec] * 4,
        out_specs=pl.BlockSpec((8, D), lambda i: (0, 0)),
        out_shape=jax.ShapeDtypeStruct((8, D), _f32),
    )(xuh, xih, xuc, xic)


def _main_body(t, xu_h, xu_c, xi_h, xi_c, m_uh, m_uc, m_ih, m_ic, degs,
               eu, ei, scv, shv,
               Wu_eh, Wu_ec, Wi_eh, Wi_ec, Wz_u, Wh_u, Wc_u, Wz_i, Wh_i, Wc_i,
               zu_enc, zi_enc, xu_th, xu_tc, xi_th, xi_tc):
    decay = jnp.exp(-t[0, 0])
    sc = scv[...]
    sh = shv[...]
    dg = degs[...]

    def bn(x, r):
        return x[...] * sc[r:r + 1, :] + sh[r:r + 1, :]

    def evo(xraw, rx, m, rg, p, W, e):
        h = (decay * bn(xraw, rx) + m[...] * sc[rg:rg + 1, :]
             + dg[:, p:p + 1] * sh[rg:rg + 1, :])
        return jnp.tanh(jnp.dot(h, W[...], preferred_element_type=_f32)) + e

    def side(xrh, xrc, rxh, rxc, mh, mc, rgh, rgc, ph, pc, e,
             Weh, Wec, Wz, Wh, Wc, zenc, xth, xtc):
        ev = e[...]
        xmh = evo(xrh, rxh, mh, rgh, ph, Weh, ev)
        xmc = evo(xrc, rxc, mc, rgc, pc, Wec, ev)

        def two(Wr):
            return (jnp.dot(xmh, Wr[pl.ds(0, D), :],
                            preferred_element_type=_f32) +
                    jnp.dot(xmc, Wr[pl.ds(D, D), :],
                            preferred_element_type=_f32))

        zenc[:, pl.ds(0, D)] = jnp.tanh(two(Wz))
        zenc[:, pl.ds(D, D)] = ev
        xth[...] = jnp.tanh(two(Wh))
        xtc[...] = jnp.tanh(two(Wc))

    side(xu_h, xu_c, 0, 2, m_uh, m_uc, 1, 3, 0, 2, eu,
         Wu_eh, Wu_ec, Wz_u, Wh_u, Wc_u, zu_enc, xu_th, xu_tc)
    side(xi_h, xi_c, 1, 3, m_ih, m_ic, 0, 2, 1, 3, ei,
         Wi_eh, Wi_ec, Wz_i, Wh_i, Wc_i, zi_enc, xi_th, xi_tc)


def _tc_main(t2, xu_h, xu_c, xi_h, xi_c, m_uh, m_uc, m_ih, m_ic, degs,
             eu, ei, scales, shifts,
             Wu_eh, Wu_ec, Wi_eh, Wi_ec, Wz_u, Wh_u, Wc_u, Wz_i, Wh_i, Wc_i):
    spec = pl.BlockSpec((_RB, D), lambda i: (i, 0))
    spec2 = pl.BlockSpec((_RB, 2 * D), lambda i: (i, 0))
    dspec = pl.BlockSpec((_RB, 4), lambda i: (i, 0))
    w1 = pl.BlockSpec((D, D), lambda i: (0, 0))
    w2 = pl.BlockSpec((2 * D, D), lambda i: (0, 0))
    cspec = pl.BlockSpec((4, D), lambda i: (0, 0))
    tspec = pl.BlockSpec((1, 1), lambda i: (0, 0))
    return pl.pallas_call(
        _main_body,
        grid=(_NB,),
        in_specs=[tspec] + [spec] * 8 + [dspec] + [spec] * 2 + [cspec] * 2
                 + [w1] * 4 + [w2] * 6,
        out_specs=[spec2, spec2, spec, spec, spec, spec],
        out_shape=[
            jax.ShapeDtypeStruct((NU, 2 * D), _f32),
            jax.ShapeDtypeStruct((NI, 2 * D), _f32),
            jax.ShapeDtypeStruct((NU, D), _f32),
            jax.ShapeDtypeStruct((NU, D), _f32),
            jax.ShapeDtypeStruct((NI, D), _f32),
            jax.ShapeDtypeStruct((NI, D), _f32),
        ],
    )(t2, xu_h, xu_c, xi_h, xi_c, m_uh, m_uc, m_ih, m_ic, degs,
      eu, ei, scales, shifts,
      Wu_eh, Wu_ec, Wi_eh, Wi_ec, Wz_u, Wh_u, Wc_u, Wz_i, Wh_i, Wc_i)


def _loss_body(zup, zip_, zun, zin, Wpu, Wpi, o):
    pu = jnp.dot(zup[...], Wpu[...], preferred_element_type=_f32)
    pi = jnp.dot(zip_[...], Wpi[...], preferred_element_type=_f32)
    cols = [jnp.sum(pu * pi, 1, keepdims=True)]
    for k in range(K):
        pik = jnp.dot(zin[:, pl.ds(k * 2 * D, 2 * D)], Wpi[...],
                      preferred_element_type=_f32)
        cols.append(jnp.sum(pu * pik, 1, keepdims=True))
    for k in range(K):
        puk = jnp.dot(zun[:, pl.ds(k * 2 * D, 2 * D)], Wpu[...],
                      preferred_element_type=_f32)
        cols.append(jnp.sum(puk * pi, 1, keepdims=True))
    scores = jnp.concatenate(cols, axis=1)
    m = jnp.max(scores, 1, keepdims=True)
    lse = jnp.log(jnp.sum(jnp.exp(scores - m), 1, keepdims=True)) + m
    loss = jnp.mean(lse[:, 0] - scores[:, 0])
    o[...] = jnp.broadcast_to(loss, (8, 128))


def _tc_loss(zu_pos, zi_pos, zu_neg2, zi_neg2, Wp_u, Wp_i):
    full = lambda s: pl.BlockSpec(s, lambda: (0, 0))
    return pl.pallas_call(
        _loss_body,
        in_specs=[full((B, 2 * D)), full((B, 2 * D)),
                  full((B, 2 * D * K)), full((B, 2 * D * K)),
                  full((2 * D, D)), full((2 * D, D))],
        out_specs=full((8, 128)),
        out_shape=jax.ShapeDtypeStruct((8, 128), _f32),
    )(zu_pos, zi_pos, zu_neg2, zi_neg2, Wp_u, Wp_i)


def _final_body(xu_th, xi_th, xu_tc, xi_tc, mu_h, mi_h, mu_c, mi_c,
                Wu_uh, Wi_uh, Wu_uc, Wi_uc, ou_h, oi_h, ou_c, oi_c):
    for x, m, w, o in ((xu_th, mu_h, Wu_uh, ou_h), (xi_th, mi_h, Wi_uh, oi_h),
                       (xu_tc, mu_c, Wu_uc, ou_c), (xi_tc, mi_c, Wi_uc, oi_c)):
        o[...] = x[...] + jnp.tanh(
            jnp.dot(m[...], w[...], preferred_element_type=_f32))


def _tc_final(xu_th, xi_th, xu_tc, xi_tc, mu_h, mi_h, mu_c, mi_c,
              Wu_uh, Wi_uh, Wu_uc, Wi_uc):
    spec = pl.BlockSpec((_RB, D), lambda i: (i, 0))
    w1 = pl.BlockSpec((D, D), lambda i: (0, 0))
    return pl.pallas_call(
        _final_body,
        grid=(_NB,),
        in_specs=[spec] * 8 + [w1] * 4,
        out_specs=[spec] * 4,
        out_shape=[jax.ShapeDtypeStruct((NU, D), _f32)] * 4,
    )(xu_th, xi_th, xu_tc, xi_tc, mu_h, mi_h, mu_c, mi_c,
      Wu_uh, Wi_uh, Wu_uc, Wi_uc)


def kernel(t_diff, adj_his, adj_ctx, adj_tgt_i2u, adj_tgt_u2i, tgt_u, tgt_i,
           tgt_u_neg, tgt_i_neg, xu_in_his, xi_in_his, xu_in_ctx, xi_in_ctx,
           embeds_u, embeds_i, g_uh, b_uh, g_ih, b_ih, g_uc, b_uc, g_ic, b_ic,
           Wu_eh, Wi_eh, Wu_ec, Wi_ec, Wu_uh, Wi_uh, Wu_uc, Wi_uc,
           Wz_u, Wh_u, Wc_u, Wz_i, Wh_i, Wc_i, Wp_u, Wp_i):
    stats = _tc_stats(xu_in_his, xi_in_his, xu_in_ctx, xi_in_ctx)
    sums = stats[0::2, :] / NU
    sqs = stats[1::2, :] / NU
    var = sqs - sums * sums
    g = jnp.stack([g_uh, g_ih, g_uc, g_ic])
    bb = jnp.stack([b_uh, b_ih, b_uc, b_ic])
    scales = g / jnp.sqrt(var + 1e-5)
    shifts = bb - sums * scales

    pad = E_PAD - E
    padsrc = (jnp.arange(pad, dtype=_i32) * 61) % NU
    padbad = NU + (jnp.arange(pad, dtype=_i32) % (ACC2 - NU))

    def prep(row):
        src = jnp.concatenate([row, padsrc]).reshape(-1, 128)
        dst = jnp.concatenate([row, padbad]).reshape(-1, 128)
        return src, dst

    s_h_s, s_h_d = prep(adj_his[0])
    d_h_s, d_h_d = prep(adj_his[1])
    s_c_s, s_c_d = prep(adj_ctx[0])
    d_c_s, d_c_d = prep(adj_ctx[1])
    zrows = jnp.zeros((STRIPE, D), _f32)
    zrows32 = jnp.zeros((WSTRIPE, HD), _f32)
    zdeg = jnp.zeros((WSTRIPE,), _f32)

    (m_uh, m_ih, m_uc, m_ic, dg_uh, dg_ih, dg_uc, dg_ic,
     _t0, _t1, _t2, _t3) = _sc_msgs(
        s_h_s, s_h_d, d_h_s, d_h_d, s_c_s, s_c_d, d_c_s, d_c_d,
        xi_in_his, xu_in_his, xi_in_ctx, xu_in_ctx, zrows32, zdeg)
    degs = jnp.stack([dg_uh, dg_ih, dg_uc, dg_ic], axis=1)

    t2 = t_diff.reshape(1, 1)
    zu_enc, zi_enc, xu_th, xu_tc, xi_th, xi_tc = _tc_main(
        t2, xu_in_his, xu_in_ctx, xi_in_his, xi_in_ctx,
        m_uh, m_uc, m_ih, m_ic, degs,
        embeds_u, embeds_i, scales, shifts,
        Wu_eh, Wu_ec, Wi_eh, Wi_ec, Wz_u, Wh_u, Wc_u, Wz_i, Wh_i, Wc_i)

    zu_pos, zi_pos, zu_neg, zi_neg, mu_h, mi_h, mu_c, mi_c = _sc_targets(
        zu_enc, zi_enc, xu_th, xi_th, xu_tc, xi_tc,
        tgt_u.reshape(-1, 128), tgt_i.reshape(-1, 128),
        tgt_u_neg.reshape(-1, 128), tgt_i_neg.reshape(-1, 128),
        adj_tgt_i2u[0].reshape(-1, 128), adj_tgt_i2u[1].reshape(-1, 128),
        adj_tgt_u2i[0].reshape(-1, 128), adj_tgt_u2i[1].reshape(-1, 128),
        zrows)

    lossmat = _tc_loss(zu_pos, zi_pos,
                       zu_neg.reshape(B, 2 * D * K),
                       zi_neg.reshape(B, 2 * D * K), Wp_u, Wp_i)
    loss = lossmat[0, 0]

    ou_h, oi_h, ou_c, oi_c = _tc_final(
        xu_th, xi_th, xu_tc, xi_tc, mu_h, mi_h, mu_c, mi_c,
        Wu_uh, Wi_uh, Wu_uc, Wi_uc)

    return (loss, zu_pos, zi_enc, ou_h, oi_h, ou_c, oi_c)


# speedup vs baseline: 7.4946x; 1.0163x over previous
import functools

import jax
import jax.numpy as jnp
from jax import lax
from jax.experimental import pallas as pl
from jax.experimental.pallas import tpu as pltpu
from jax.experimental.pallas import tpu_sc as plsc

NU = 50000
NI = 50000
D = 64
E = 800000
B = 4096
K = 5

NC = 2
NS = 16
HALF = NU // NC
ACC_ROWS = 25088
STRIPE = 1568
LAST_STRIPE = HALF - (NS - 1) * STRIPE
E_PAD = 802816
NCHUNK = 49

_f32 = jnp.float32
_i32 = jnp.int32


HD = D // 2
ACC2 = 50048
WSTRIPE = 3128
WLAST = NU - (NS - 1) * WSTRIPE


def _sc_edge_phase(srcr, dstr, tab, outr, degr, do_deg, zrows, zdeg,
                   src_v, dst_v, src_v1, dst_v1, rows, ones_v, acc, dacc,
                   gsem, ssem, isem, dsem, cid, sid):
    pltpu.sync_copy(zrows, acc.at[pl.ds(sid * WSTRIPE, WSTRIPE)])
    pltpu.sync_copy(zdeg, dacc.at[pl.ds(sid * WSTRIPE, WSTRIPE)])
    plsc.subcore_barrier()

    for cc in range(NC):
        @pl.when(cid == cc)
        def _(cc=cc):
            tabh = tab.at[cc]
            my_deg = do_deg == cc

            def proc(sv, dv, i):
                gd = [None] * 8
                sd = [None] * 8
                dd = [None] * 8
                gd[0] = pltpu.async_copy(tabh.at[sv.at[0]],
                                         rows.at[pl.ds(0, 128)], gsem)
                for j in range(8):
                    if j + 1 < 8:
                        if j >= 4:
                            sd[j - 4].wait()
                        gd[j + 1] = pltpu.async_copy(
                            tabh.at[sv.at[j + 1]],
                            rows.at[pl.ds(((j + 1) % 5) * 128, 128)], gsem)
                    gd[j].wait()
                    sd[j] = pltpu.async_copy(
                        rows.at[pl.ds((j % 5) * 128, 128)],
                        acc.at[dv.at[j]], ssem, add=True)
                    if my_deg:
                        dd[j] = pltpu.async_copy(
                            ones_v, dacc.at[dv.at[j]], dsem, add=True)
                for j in range(3, 8):
                    sd[j].wait()
                if my_deg:
                    for j in range(8):
                        dd[j].wait()

            def erow(i):
                return sid * (NCHUNK * 8) + i * 8

            pltpu.sync_copy(srcr.at[pl.ds(erow(0), 8)], src_v)
            pltpu.sync_copy(dstr.at[pl.ds(erow(0), 8)], dst_v)

            def pair(p, carry):
                i0 = 2 * p
                dA = pltpu.async_copy(srcr.at[pl.ds(erow(i0 + 1), 8)],
                                      src_v1, isem)
                dB = pltpu.async_copy(dstr.at[pl.ds(erow(i0 + 1), 8)],
                                      dst_v1, isem)
                proc(src_v, dst_v, i0)
                dA.wait()
                dB.wait()
                dC = pltpu.async_copy(srcr.at[pl.ds(erow(i0 + 2), 8)],
                                      src_v, isem)
                dD = pltpu.async_copy(dstr.at[pl.ds(erow(i0 + 2), 8)],
                                      dst_v, isem)
                proc(src_v1, dst_v1, i0 + 1)
                dC.wait()
                dD.wait()
                return carry

            lax.fori_loop(0, (NCHUNK - 1) // 2, pair, 0)
            proc(src_v, dst_v, NCHUNK - 1)

    plsc.subcore_barrier()

    nrows = [WSTRIPE, WLAST]
    for cc in range(NC):
        for last in range(2):
            sel = (sid == NS - 1) if last else (sid < NS - 1)
            r0c = (NS - 1) * WSTRIPE if last else None

            @pl.when((cid == cc) & sel)
            def _(cc=cc, n=nrows[last], last=last):
                r0 = (NS - 1) * WSTRIPE if last else sid * WSTRIPE
                pltpu.sync_copy(acc.at[pl.ds(r0, n)],
                                outr.at[pl.ds(r0, n), pl.ds(cc * HD, HD)])

            if do_deg == cc:
                @pl.when((cid == cc) & sel)
                def _(n=nrows[last], last=last):
                    r0 = (NS - 1) * WSTRIPE if last else sid * WSTRIPE
                    pltpu.sync_copy(dacc.at[pl.ds(r0, n)],
                                    degr.at[pl.ds(r0, n)])


def _sc_msgs_body(s_h_s, s_h_d, d_h_s, d_h_d, s_c_s, s_c_d, d_c_s, d_c_d,
                  r_ih, r_uh, r_ic, r_uc, zrows, zdeg,
                  o_uh, o_ih, o_uc, o_ic, dg_uh, dg_ih, dg_uc, dg_ic,
                  tb_ih, tb_uh, tb_ic, tb_uc,
                  src_v, dst_v, src_v1, dst_v1, rows, ones_v, acc, dacc,
                  gsem, ssem, isem, dsem):
    cid = lax.axis_index("c")
    sid = lax.axis_index("s")
    for t in range(8):
        ones_v[pl.ds(t * 16, 16)] = jnp.full((16,), 1.0, _f32)

    for raw, tb in ((r_ih, tb_ih), (r_uh, tb_uh), (r_ic, tb_ic),
                    (r_uc, tb_uc)):
        for cc in range(NC):
            @pl.when(cid == cc)
            def _(raw=raw, tb=tb, cc=cc):
                base = sid * WSTRIPE

                def cp(r0, n):
                    pltpu.sync_copy(
                        raw.at[pl.ds(r0, n), pl.ds(cc * HD, HD)],
                        rows.at[pl.ds(0, n)])
                    pltpu.sync_copy(rows.at[pl.ds(0, n)],
                                    tb.at[cc].at[pl.ds(r0, n)])

                def q4(q, carry):
                    cp(base + q * 640, 640)
                    return carry

                lax.fori_loop(0, 4, q4, 0)

                @pl.when(sid < NS - 1)
                def _():
                    cp(base + 2560, WSTRIPE - 2560)

                @pl.when(sid == NS - 1)
                def _():
                    cp(base + 2560, WLAST - 2560)
    plsc.subcore_barrier()

    phases = (
        (d_h_s, s_h_d, tb_ih, o_uh, dg_uh, 0),
        (s_h_s, d_h_d, tb_uh, o_ih, dg_ih, 1),
        (d_c_s, s_c_d, tb_ic, o_uc, dg_uc, 0),
        (s_c_s, d_c_d, tb_uc, o_ic, dg_ic, 1),
    )
    for srcr, dstr, tab, outr, degr, do_deg in phases:
        _sc_edge_phase(srcr, dstr, tab, outr, degr, do_deg, zrows, zdeg,
                       src_v, dst_v, src_v1, dst_v1, rows, ones_v, acc,
                       dacc, gsem, ssem, isem, dsem, cid, sid)


_sc_msgs = functools.partial(
    pl.kernel,
    out_type=tuple(
        [jax.ShapeDtypeStruct((NU, D), _f32) for _ in range(4)]
        + [jax.ShapeDtypeStruct((NU,), _f32) for _ in range(4)]
        + [jax.ShapeDtypeStruct((NC, NU, HD), _f32) for _ in range(4)]),
    mesh=plsc.VectorSubcoreMesh(core_axis_name="c", subcore_axis_name="s"),
    scratch_types=[
        pltpu.VMEM((8, 128), _i32),
        pltpu.VMEM((8, 128), _i32),
        pltpu.VMEM((8, 128), _i32),
        pltpu.VMEM((8, 128), _i32),
        pltpu.VMEM((640, HD), _f32),
        pltpu.VMEM((128,), _f32),
        pltpu.VMEM_SHARED((ACC2, HD), _f32),
        pltpu.VMEM_SHARED((ACC2,), _f32),
        pltpu.SemaphoreType.DMA,
        pltpu.SemaphoreType.DMA,
        pltpu.SemaphoreType.DMA,
        pltpu.SemaphoreType.DMA,
    ],
    compiler_params=pltpu.CompilerParams(use_tc_tiling_on_sc=False),
)(_sc_msgs_body)


def _sc_targets_body(zu_enc, zi_enc, xu_th, xi_th, xu_tc, xi_tc,
                     tgtu, tgti, tgtun, tgtin,
                     i2u_s, i2u_d, u2i_s, u2i_d, zrows,
                     zu_pos, zi_pos, zu_neg, zi_neg, mu_u, mu_i,
                     idx_v, rows128, rows64, ldst, acc, gsem, ssem):
    cid = lax.axis_index("c")
    sid = lax.axis_index("s")
    wid = sid * NC + cid

    for idx2d, tab, outr, n_per_w in (
            (tgtu, zu_enc, zu_pos, 1),
            (tgti, zi_enc, zi_pos, 1),
            (tgtun, zu_enc, zu_neg, 5),
            (tgtin, zi_enc, zi_neg, 5),
    ):
        for q in range(n_per_w):
            crow = wid * n_per_w + q
            pltpu.sync_copy(idx2d.at[pl.ds(crow, 1)], idx_v.at[pl.ds(0, 1)])
            pltpu.async_copy(tab.at[idx_v.at[0]], rows128, gsem).wait()
            pltpu.sync_copy(rows128, outr.at[pl.ds(crow * 128, 128)])

    base_row = cid * HALF
    trash = HALF + 8 + sid * 4
    for src2d, dst2d, tab, outr, cb in (
            (i2u_s, i2u_d, xi_th, mu_u, 0),
            (u2i_s, u2i_d, xu_th, mu_i, 0),
            (i2u_s, i2u_d, xi_tc, mu_u, D),
            (u2i_s, u2i_d, xu_tc, mu_i, D),
    ):
        pltpu.sync_copy(zrows, acc.at[pl.ds(sid * STRIPE, STRIPE)])
        plsc.subcore_barrier()
        for q in range(2):
            crow = sid * 2 + q
            pltpu.sync_copy(src2d.at[pl.ds(crow, 1)], idx_v.at[pl.ds(0, 1)])
            pltpu.sync_copy(dst2d.at[pl.ds(crow, 1)], idx_v.at[pl.ds(1, 1)])
            dsc = pltpu.async_copy(tab.at[idx_v.at[0]], rows64, gsem)
            for t in range(8):
                v = idx_v[1, pl.ds(t * 16, 16)]
                tl = v - base_row
                m = (tl >= 0) & (tl < HALF)
                ldst[0, pl.ds(t * 16, 16)] = jnp.where(m, tl, trash)
            dsc.wait()
            pltpu.async_copy(rows64, acc.at[ldst.at[0]], ssem,
                             add=True).wait()
        plsc.subcore_barrier()

        @pl.when(sid < NS - 1)
        def _(outr=outr, cb=cb):
            pltpu.sync_copy(acc.at[pl.ds(sid * STRIPE, STRIPE)],
                            outr.at[pl.ds(base_row + sid * STRIPE, STRIPE),
                                    pl.ds(cb, D)])

        @pl.when(sid == NS - 1)
        def _(outr=outr, cb=cb):
            pltpu.sync_copy(acc.at[pl.ds((NS - 1) * STRIPE, LAST_STRIPE)],
                            outr.at[pl.ds(base_row + (NS - 1) * STRIPE,
                                          LAST_STRIPE), pl.ds(cb, D)])


_sc_targets = functools.partial(
    pl.kernel,
    out_type=(
        jax.ShapeDtypeStruct((B, 2 * D), _f32),
        jax.ShapeDtypeStruct((B, 2 * D), _f32),
        jax.ShapeDtypeStruct((B * K, 2 * D), _f32),
        jax.ShapeDtypeStruct((B * K, 2 * D), _f32),
        jax.ShapeDtypeStruct((NU, 2 * D), _f32),
        jax.ShapeDtypeStruct((NI, 2 * D), _f32),
    ),
    mesh=plsc.VectorSubcoreMesh(core_axis_name="c", subcore_axis_name="s"),
    scratch_types=[
        pltpu.VMEM((8, 128), _i32),
        pltpu.VMEM((128, 2 * D), _f32),
        pltpu.VMEM((128, D), _f32),
        pltpu.VMEM((8, 128), _i32),
        pltpu.VMEM_SHARED((ACC_ROWS, D), _f32),
        pltpu.SemaphoreType.DMA,
        pltpu.SemaphoreType.DMA,
    ],
    compiler_params=pltpu.CompilerParams(use_tc_tiling_on_sc=False),
)(_sc_targets_body)


_RB = 1000
_NB = NU // _RB


def _stats_body(a, b, c, d, o):
    i = pl.program_id(0)

    @pl.when(i == 0)
    def _():
        o[...] = jnp.zeros_like(o)

    for r, x in enumerate((a, b, c, d)):
        xv = x[...]
        o[pl.ds(2 * r, 1), :] += jnp.sum(xv, 0, keepdims=True)
        o[pl.ds(2 * r + 1, 1), :] += jnp.sum(xv * xv, 0, keepdims=True)


def _tc_stats(xuh, xih, xuc, xic):
    spec = pl.BlockSpec((_RB, D), lambda i: (i, 0))
    return pl.pallas_call(
        _stats_body,
        grid=(_NB,),
        in_specs=[spec] * 4,
        out_specs=pl.BlockSpec((8, D), lambda i: (0, 0)),
        out_shape=jax.ShapeDtypeStruct((8, D), _f32),
    )(xuh, xih, xuc, xic)


def _main_body(t, xu_h, xu_c, xi_h, xi_c, m_uh, m_uc, m_ih, m_ic, degs,
               eu, ei, scv, shv,
               Wu_eh, Wu_ec, Wi_eh, Wi_ec, Wz_u, Wh_u, Wc_u, Wz_i, Wh_i, Wc_i,
               zu_enc, zi_enc, xu_t, xi_t):
    decay = jnp.exp(-t[0, 0])
    sc = scv[...]
    sh = shv[...]
    dg = degs[...]

    def bn(x, r):
        return x[...] * sc[r:r + 1, :] + sh[r:r + 1, :]

    def evo(xraw, rx, m, rg, p, W, e):
        h = (decay * bn(xraw, rx) + m[...] * sc[rg:rg + 1, :]
             + dg[:, p:p + 1] * sh[rg:rg + 1, :])
        return jnp.tanh(jnp.dot(h, W[...], preferred_element_type=_f32)) + e

    def side(xrh, xrc, rxh, rxc, mh, mc, rgh, rgc, ph, pc, e,
             Weh, Wec, Wz, Wh, Wc, zenc, xt):
        ev = e[...]
        xmh = evo(xrh, rxh, mh, rgh, ph, Weh, ev)
        xmc = evo(xrc, rxc, mc, rgc, pc, Wec, ev)

        def two(Wr):
            return (jnp.dot(xmh, Wr[pl.ds(0, D), :],
                            preferred_element_type=_f32) +
                    jnp.dot(xmc, Wr[pl.ds(D, D), :],
                            preferred_element_type=_f32))

        zenc[:, pl.ds(0, D)] = jnp.tanh(two(Wz))
        zenc[:, pl.ds(D, D)] = ev
        xt[:, pl.ds(0, D)] = jnp.tanh(two(Wh))
        xt[:, pl.ds(D, D)] = jnp.tanh(two(Wc))

    side(xu_h, xu_c, 0, 2, m_uh, m_uc, 1, 3, 0, 2, eu,
         Wu_eh, Wu_ec, Wz_u, Wh_u, Wc_u, zu_enc, xu_t)
    side(xi_h, xi_c, 1, 3, m_ih, m_ic, 0, 2, 1, 3, ei,
         Wi_eh, Wi_ec, Wz_i, Wh_i, Wc_i, zi_enc, xi_t)


def _tc_main(t2, xu_h, xu_c, xi_h, xi_c, m_uh, m_uc, m_ih, m_ic, degs,
             eu, ei, scales, shifts,
             Wu_eh, Wu_ec, Wi_eh, Wi_ec, Wz_u, Wh_u, Wc_u, Wz_i, Wh_i, Wc_i):
    spec = pl.BlockSpec((_RB, D), lambda i: (i, 0))
    spec2 = pl.BlockSpec((_RB, 2 * D), lambda i: (i, 0))
    dspec = pl.BlockSpec((_RB, 4), lambda i: (i, 0))
    w1 = pl.BlockSpec((D, D), lambda i: (0, 0))
    w2 = pl.BlockSpec((2 * D, D), lambda i: (0, 0))
    cspec = pl.BlockSpec((4, D), lambda i: (0, 0))
    tspec = pl.BlockSpec((1, 1), lambda i: (0, 0))
    return pl.pallas_call(
        _main_body,
        grid=(_NB,),
        in_specs=[tspec] + [spec] * 8 + [dspec] + [spec] * 2 + [cspec] * 2
                 + [w1] * 4 + [w2] * 6,
        out_specs=[spec2, spec2, spec2, spec2],
        out_shape=[
            jax.ShapeDtypeStruct((NU, 2 * D), _f32),
            jax.ShapeDtypeStruct((NI, 2 * D), _f32),
            jax.ShapeDtypeStruct((NU, 2 * D), _f32),
            jax.ShapeDtypeStruct((NI, 2 * D), _f32),
        ],
    )(t2, xu_h, xu_c, xi_h, xi_c, m_uh, m_uc, m_ih, m_ic, degs,
      eu, ei, scales, shifts,
      Wu_eh, Wu_ec, Wi_eh, Wi_ec, Wz_u, Wh_u, Wc_u, Wz_i, Wh_i, Wc_i)


def _loss_body(zup, zip_, zun, zin, Wpu, Wpi, o):
    pu = jnp.dot(zup[...], Wpu[...], preferred_element_type=_f32)
    pi = jnp.dot(zip_[...], Wpi[...], preferred_element_type=_f32)
    cols = [jnp.sum(pu * pi, 1, keepdims=True)]
    for k in range(K):
        pik = jnp.dot(zin[:, pl.ds(k * 2 * D, 2 * D)], Wpi[...],
                      preferred_element_type=_f32)
        cols.append(jnp.sum(pu * pik, 1, keepdims=True))
    for k in range(K):
        puk = jnp.dot(zun[:, pl.ds(k * 2 * D, 2 * D)], Wpu[...],
                      preferred_element_type=_f32)
        cols.append(jnp.sum(puk * pi, 1, keepdims=True))
    scores = jnp.concatenate(cols, axis=1)
    m = jnp.max(scores, 1, keepdims=True)
    lse = jnp.log(jnp.sum(jnp.exp(scores - m), 1, keepdims=True)) + m
    loss = jnp.mean(lse[:, 0] - scores[:, 0])
    o[...] = jnp.broadcast_to(loss, (8, 128))


def _tc_loss(zu_pos, zi_pos, zu_neg2, zi_neg2, Wp_u, Wp_i):
    full = lambda s: pl.BlockSpec(s, lambda: (0, 0))
    return pl.pallas_call(
        _loss_body,
        in_specs=[full((B, 2 * D)), full((B, 2 * D)),
                  full((B, 2 * D * K)), full((B, 2 * D * K)),
                  full((2 * D, D)), full((2 * D, D))],
        out_specs=full((8, 128)),
        out_shape=jax.ShapeDtypeStruct((8, 128), _f32),
    )(zu_pos, zi_pos, zu_neg2, zi_neg2, Wp_u, Wp_i)


def _final_body(xu_t, xi_t, mu_u, mu_i,
                Wu_uh, Wi_uh, Wu_uc, Wi_uc, ou_h, oi_h, ou_c, oi_c):
    xu = xu_t[...]
    xi = xi_t[...]
    mu = mu_u[...]
    mi = mu_i[...]
    for x, m, w, o in (
            (xu[:, :D], mu[:, :D], Wu_uh, ou_h),
            (xi[:, :D], mi[:, :D], Wi_uh, oi_h),
            (xu[:, D:], mu[:, D:], Wu_uc, ou_c),
            (xi[:, D:], mi[:, D:], Wi_uc, oi_c)):
        o[...] = x + jnp.tanh(
            jnp.dot(m, w[...], preferred_element_type=_f32))


def _tc_final(xu_t, xi_t, mu_u, mu_i, Wu_uh, Wi_uh, Wu_uc, Wi_uc):
    spec = pl.BlockSpec((_RB, D), lambda i: (i, 0))
    spec2 = pl.BlockSpec((_RB, 2 * D), lambda i: (i, 0))
    w1 = pl.BlockSpec((D, D), lambda i: (0, 0))
    return pl.pallas_call(
        _final_body,
        grid=(_NB,),
        in_specs=[spec2] * 4 + [w1] * 4,
        out_specs=[spec] * 4,
        out_shape=[jax.ShapeDtypeStruct((NU, D), _f32)] * 4,
    )(xu_t, xi_t, mu_u, mu_i, Wu_uh, Wi_uh, Wu_uc, Wi_uc)


def kernel(t_diff, adj_his, adj_ctx, adj_tgt_i2u, adj_tgt_u2i, tgt_u, tgt_i,
           tgt_u_neg, tgt_i_neg, xu_in_his, xi_in_his, xu_in_ctx, xi_in_ctx,
           embeds_u, embeds_i, g_uh, b_uh, g_ih, b_ih, g_uc, b_uc, g_ic, b_ic,
           Wu_eh, Wi_eh, Wu_ec, Wi_ec, Wu_uh, Wi_uh, Wu_uc, Wi_uc,
           Wz_u, Wh_u, Wc_u, Wz_i, Wh_i, Wc_i, Wp_u, Wp_i):
    stats = _tc_stats(xu_in_his, xi_in_his, xu_in_ctx, xi_in_ctx)
    sums = stats[0::2, :] / NU
    sqs = stats[1::2, :] / NU
    var = sqs - sums * sums
    g = jnp.stack([g_uh, g_ih, g_uc, g_ic])
    bb = jnp.stack([b_uh, b_ih, b_uc, b_ic])
    scales = g / jnp.sqrt(var + 1e-5)
    shifts = bb - sums * scales

    pad = E_PAD - E
    padsrc = (jnp.arange(pad, dtype=_i32) * 61) % NU
    padbad = NU + (jnp.arange(pad, dtype=_i32) % (ACC2 - NU))

    def prep(row):
        src = jnp.concatenate([row, padsrc]).reshape(-1, 128)
        dst = jnp.concatenate([row, padbad]).reshape(-1, 128)
        return src, dst

    s_h_s, s_h_d = prep(adj_his[0])
    d_h_s, d_h_d = prep(adj_his[1])
    s_c_s, s_c_d = prep(adj_ctx[0])
    d_c_s, d_c_d = prep(adj_ctx[1])
    zrows = jnp.zeros((STRIPE, D), _f32)
    zrows32 = jnp.zeros((WSTRIPE, HD), _f32)
    zdeg = jnp.zeros((WSTRIPE,), _f32)

    (m_uh, m_ih, m_uc, m_ic, dg_uh, dg_ih, dg_uc, dg_ic,
     _t0, _t1, _t2, _t3) = _sc_msgs(
        s_h_s, s_h_d, d_h_s, d_h_d, s_c_s, s_c_d, d_c_s, d_c_d,
        xi_in_his, xu_in_his, xi_in_ctx, xu_in_ctx, zrows32, zdeg)
    degs = jnp.stack([dg_uh, dg_ih, dg_uc, dg_ic], axis=1)

    t2 = t_diff.reshape(1, 1)
    zu_enc, zi_enc, xu_t, xi_t = _tc_main(
        t2, xu_in_his, xu_in_ctx, xi_in_his, xi_in_ctx,
        m_uh, m_uc, m_ih, m_ic, degs,
        embeds_u, embeds_i, scales, shifts,
        Wu_eh, Wu_ec, Wi_eh, Wi_ec, Wz_u, Wh_u, Wc_u, Wz_i, Wh_i, Wc_i)
    xu_th, xu_tc = xu_t[:, :D], xu_t[:, D:]
    xi_th, xi_tc = xi_t[:, :D], xi_t[:, D:]

    zu_pos, zi_pos, zu_neg, zi_neg, mu_u, mu_i = _sc_targets(
        zu_enc, zi_enc, xu_th, xi_th, xu_tc, xi_tc,
        tgt_u.reshape(-1, 128), tgt_i.reshape(-1, 128),
        tgt_u_neg.reshape(-1, 128), tgt_i_neg.reshape(-1, 128),
        adj_tgt_i2u[0].reshape(-1, 128), adj_tgt_i2u[1].reshape(-1, 128),
        adj_tgt_u2i[0].reshape(-1, 128), adj_tgt_u2i[1].reshape(-1, 128),
        zrows)

    lossmat = _tc_loss(zu_pos, zi_pos,
                       zu_neg.reshape(B, 2 * D * K),
                       zi_neg.reshape(B, 2 * D * K), Wp_u, Wp_i)
    loss = lossmat[0, 0]

    ou_h, oi_h, ou_c, oi_c = _tc_final(
        xu_t, xi_t, mu_u, mu_i, Wu_uh, Wi_uh, Wu_uc, Wi_uc)

    return (loss, zu_pos, zi_enc, ou_h, oi_h, ou_c, oi_c)


# speedup vs baseline: 7.6696x; 1.0233x over previous
import functools

import jax
import jax.numpy as jnp
from jax import lax
from jax.experimental import pallas as pl
from jax.experimental.pallas import tpu as pltpu
from jax.experimental.pallas import tpu_sc as plsc

NU = 50000
NI = 50000
D = 64
E = 800000
B = 4096
K = 5

NC = 2
NS = 16
HALF = NU // NC
ACC_ROWS = 25088
STRIPE = 1568
LAST_STRIPE = HALF - (NS - 1) * STRIPE
E_PAD = 802816
NCHUNK = 49

_f32 = jnp.float32
_i32 = jnp.int32


HD = D // 2
ACC2 = 50048
WSTRIPE = 3128
WLAST = NU - (NS - 1) * WSTRIPE


def _sc_edge_phase(srcr, dstr, tab, outr, degr, do_deg, zrows, zdeg,
                   src_v, dst_v, src_v1, dst_v1, rows, ones_v, acc, dacc,
                   gsem, ssem, isem, dsem, cid, sid):
    pltpu.sync_copy(zrows, acc.at[pl.ds(sid * WSTRIPE, WSTRIPE)])
    pltpu.sync_copy(zdeg, dacc.at[pl.ds(sid * WSTRIPE, WSTRIPE)])
    plsc.subcore_barrier()

    for cc in range(NC):
        @pl.when(cid == cc)
        def _(cc=cc):
            tabh = tab.at[cc]
            my_deg = do_deg == cc

            def proc(sv, dv, i):
                gd = [None] * 8
                sd = [None] * 8
                dd = [None] * 8
                gd[0] = pltpu.async_copy(tabh.at[sv.at[0]],
                                         rows.at[pl.ds(0, 128)], gsem)
                for j in range(8):
                    if j + 1 < 8:
                        if j >= 4:
                            sd[j - 4].wait()
                        gd[j + 1] = pltpu.async_copy(
                            tabh.at[sv.at[j + 1]],
                            rows.at[pl.ds(((j + 1) % 5) * 128, 128)], gsem)
                    gd[j].wait()
                    sd[j] = pltpu.async_copy(
                        rows.at[pl.ds((j % 5) * 128, 128)],
                        acc.at[dv.at[j]], ssem, add=True)
                    if my_deg:
                        dd[j] = pltpu.async_copy(
                            ones_v, dacc.at[dv.at[j]], dsem, add=True)
                for j in range(3, 8):
                    sd[j].wait()
                if my_deg:
                    for j in range(8):
                        dd[j].wait()

            def erow(i):
                return sid * (NCHUNK * 8) + i * 8

            pltpu.sync_copy(srcr.at[pl.ds(erow(0), 8)], src_v)
            pltpu.sync_copy(dstr.at[pl.ds(erow(0), 8)], dst_v)

            def pair(p, carry):
                i0 = 2 * p
                dA = pltpu.async_copy(srcr.at[pl.ds(erow(i0 + 1), 8)],
                                      src_v1, isem)
                dB = pltpu.async_copy(dstr.at[pl.ds(erow(i0 + 1), 8)],
                                      dst_v1, isem)
                proc(src_v, dst_v, i0)
                dA.wait()
                dB.wait()
                dC = pltpu.async_copy(srcr.at[pl.ds(erow(i0 + 2), 8)],
                                      src_v, isem)
                dD = pltpu.async_copy(dstr.at[pl.ds(erow(i0 + 2), 8)],
                                      dst_v, isem)
                proc(src_v1, dst_v1, i0 + 1)
                dC.wait()
                dD.wait()
                return carry

            lax.fori_loop(0, (NCHUNK - 1) // 2, pair, 0)
            proc(src_v, dst_v, NCHUNK - 1)

    plsc.subcore_barrier()

    nrows = [WSTRIPE, WLAST]
    for cc in range(NC):
        for last in range(2):
            sel = (sid == NS - 1) if last else (sid < NS - 1)

            @pl.when((cid == cc) & sel)
            def _(cc=cc, n=nrows[last], last=last):
                r0 = (NS - 1) * WSTRIPE if last else sid * WSTRIPE
                pltpu.sync_copy(acc.at[pl.ds(r0, n)],
                                outr.at[pl.ds(r0, n), pl.ds(cc * HD, HD)])

            if do_deg == cc:
                @pl.when((cid == cc) & sel)
                def _(n=nrows[last], last=last):
                    r0 = (NS - 1) * WSTRIPE if last else sid * WSTRIPE
                    pltpu.sync_copy(dacc.at[pl.ds(r0, n)],
                                    degr.at[pl.ds(r0, n)])


def _sc_msgs_body(s_h_s, s_h_d, d_h_s, d_h_d, s_c_s, s_c_d, d_c_s, d_c_d,
                  r_ih, r_uh, r_ic, r_uc, zrows, zdeg,
                  o_uh, o_ih, o_uc, o_ic, dg_uh, dg_ih, dg_uc, dg_ic,
                  tb_ih, tb_uh, tb_ic, tb_uc,
                  src_v, dst_v, src_v1, dst_v1, rows, ones_v, acc, dacc,
                  gsem, ssem, isem, dsem):
    cid = lax.axis_index("c")
    sid = lax.axis_index("s")
    for t in range(8):
        ones_v[pl.ds(t * 16, 16)] = jnp.full((16,), 1.0, _f32)

    for raw, tb in ((r_ih, tb_ih), (r_uh, tb_uh), (r_ic, tb_ic),
                    (r_uc, tb_uc)):
        for cc in range(NC):
            @pl.when(cid == cc)
            def _(raw=raw, tb=tb, cc=cc):
                base = sid * WSTRIPE

                def cp(r0, n):
                    pltpu.sync_copy(
                        raw.at[pl.ds(r0, n), pl.ds(cc * HD, HD)],
                        rows.at[pl.ds(0, n)])
                    pltpu.sync_copy(rows.at[pl.ds(0, n)],
                                    tb.at[cc].at[pl.ds(r0, n)])

                def q4(q, carry):
                    cp(base + q * 640, 640)
                    return carry

                lax.fori_loop(0, 4, q4, 0)

                @pl.when(sid < NS - 1)
                def _():
                    cp(base + 2560, WSTRIPE - 2560)

                @pl.when(sid == NS - 1)
                def _():
                    cp(base + 2560, WLAST - 2560)
    plsc.subcore_barrier()

    phases = (
        (d_h_s, s_h_d, tb_ih, o_uh, dg_uh, 0),
        (s_h_s, d_h_d, tb_uh, o_ih, dg_ih, 1),
        (d_c_s, s_c_d, tb_ic, o_uc, dg_uc, 0),
        (s_c_s, d_c_d, tb_uc, o_ic, dg_ic, 1),
    )
    for srcr, dstr, tab, outr, degr, do_deg in phases:
        _sc_edge_phase(srcr, dstr, tab, outr, degr, do_deg, zrows, zdeg,
                       src_v, dst_v, src_v1, dst_v1, rows, ones_v, acc,
                       dacc, gsem, ssem, isem, dsem, cid, sid)


_sc_msgs = functools.partial(
    pl.kernel,
    out_type=tuple(
        [jax.ShapeDtypeStruct((NU, D), _f32) for _ in range(4)]
        + [jax.ShapeDtypeStruct((NU,), _f32) for _ in range(4)]
        + [jax.ShapeDtypeStruct((NC, NU, HD), _f32) for _ in range(4)]),
    mesh=plsc.VectorSubcoreMesh(core_axis_name="c", subcore_axis_name="s"),
    scratch_types=[
        pltpu.VMEM((8, 128), _i32),
        pltpu.VMEM((8, 128), _i32),
        pltpu.VMEM((8, 128), _i32),
        pltpu.VMEM((8, 128), _i32),
        pltpu.VMEM((640, HD), _f32),
        pltpu.VMEM((128,), _f32),
        pltpu.VMEM_SHARED((ACC2, HD), _f32),
        pltpu.VMEM_SHARED((ACC2,), _f32),
        pltpu.SemaphoreType.DMA,
        pltpu.SemaphoreType.DMA,
        pltpu.SemaphoreType.DMA,
        pltpu.SemaphoreType.DMA,
    ],
    compiler_params=pltpu.CompilerParams(use_tc_tiling_on_sc=False),
)(_sc_msgs_body)


def _sc_targets_body(zu_enc, zi_enc, xu_th, xi_th, xu_tc, xi_tc,
                     tgtu, tgti, tgtun, tgtin,
                     i2u_s, i2u_d, u2i_s, u2i_d, zrows,
                     zu_pos, zi_pos, zu_neg, zi_neg, mu_u, mu_i,
                     idx_v, rows128, rows64, ldst, acc, gsem, ssem):
    cid = lax.axis_index("c")
    sid = lax.axis_index("s")
    wid = sid * NC + cid

    for idx2d, tab, outr, n_per_w in (
            (tgtu, zu_enc, zu_pos, 1),
            (tgti, zi_enc, zi_pos, 1),
            (tgtun, zu_enc, zu_neg, 5),
            (tgtin, zi_enc, zi_neg, 5),
    ):
        for q in range(n_per_w):
            crow = wid * n_per_w + q
            pltpu.sync_copy(idx2d.at[pl.ds(crow, 1)], idx_v.at[pl.ds(0, 1)])
            pltpu.async_copy(tab.at[idx_v.at[0]], rows128, gsem).wait()
            pltpu.sync_copy(rows128, outr.at[pl.ds(crow * 128, 128)])

    base_row = cid * HALF
    trash = HALF + 8 + sid * 4
    for src2d, dst2d, tab, outr, cb in (
            (i2u_s, i2u_d, xi_th, mu_u, 0),
            (u2i_s, u2i_d, xu_th, mu_i, 0),
            (i2u_s, i2u_d, xi_tc, mu_u, D),
            (u2i_s, u2i_d, xu_tc, mu_i, D),
    ):
        pltpu.sync_copy(zrows, acc.at[pl.ds(sid * STRIPE, STRIPE)])
        plsc.subcore_barrier()
        for q in range(2):
            crow = sid * 2 + q
            pltpu.sync_copy(src2d.at[pl.ds(crow, 1)], idx_v.at[pl.ds(0, 1)])
            pltpu.sync_copy(dst2d.at[pl.ds(crow, 1)], idx_v.at[pl.ds(1, 1)])
            dsc = pltpu.async_copy(tab.at[idx_v.at[0]], rows64, gsem)
            for t in range(8):
                v = idx_v[1, pl.ds(t * 16, 16)]
                tl = v - base_row
                m = (tl >= 0) & (tl < HALF)
                ldst[0, pl.ds(t * 16, 16)] = jnp.where(m, tl, trash)
            dsc.wait()
            pltpu.async_copy(rows64, acc.at[ldst.at[0]], ssem,
                             add=True).wait()
        plsc.subcore_barrier()

        @pl.when(sid < NS - 1)
        def _(outr=outr, cb=cb):
            pltpu.sync_copy(acc.at[pl.ds(sid * STRIPE, STRIPE)],
                            outr.at[pl.ds(base_row + sid * STRIPE, STRIPE),
                                    pl.ds(cb, D)])

        @pl.when(sid == NS - 1)
        def _(outr=outr, cb=cb):
            pltpu.sync_copy(acc.at[pl.ds((NS - 1) * STRIPE, LAST_STRIPE)],
                            outr.at[pl.ds(base_row + (NS - 1) * STRIPE,
                                          LAST_STRIPE), pl.ds(cb, D)])


_sc_targets = functools.partial(
    pl.kernel,
    out_type=(
        jax.ShapeDtypeStruct((B, 2 * D), _f32),
        jax.ShapeDtypeStruct((B, 2 * D), _f32),
        jax.ShapeDtypeStruct((B * K, 2 * D), _f32),
        jax.ShapeDtypeStruct((B * K, 2 * D), _f32),
        jax.ShapeDtypeStruct((NU, 2 * D), _f32),
        jax.ShapeDtypeStruct((NI, 2 * D), _f32),
    ),
    mesh=plsc.VectorSubcoreMesh(core_axis_name="c", subcore_axis_name="s"),
    scratch_types=[
        pltpu.VMEM((8, 128), _i32),
        pltpu.VMEM((128, 2 * D), _f32),
        pltpu.VMEM((128, D), _f32),
        pltpu.VMEM((8, 128), _i32),
        pltpu.VMEM_SHARED((ACC_ROWS, D), _f32),
        pltpu.SemaphoreType.DMA,
        pltpu.SemaphoreType.DMA,
    ],
    compiler_params=pltpu.CompilerParams(use_tc_tiling_on_sc=False),
)(_sc_targets_body)


_RB = 2000
_NB = NU // _RB


def _stats_body(a, b, c, d, o):
    i = pl.program_id(0)

    @pl.when(i == 0)
    def _():
        o[...] = jnp.zeros_like(o)

    for r, x in enumerate((a, b, c, d)):
        xv = x[...]
        o[pl.ds(2 * r, 1), :] += jnp.sum(xv, 0, keepdims=True)
        o[pl.ds(2 * r + 1, 1), :] += jnp.sum(xv * xv, 0, keepdims=True)


def _tc_stats(xuh, xih, xuc, xic):
    spec = pl.BlockSpec((_RB, D), lambda i: (i, 0))
    return pl.pallas_call(
        _stats_body,
        grid=(_NB,),
        in_specs=[spec] * 4,
        out_specs=pl.BlockSpec((8, D), lambda i: (0, 0)),
        out_shape=jax.ShapeDtypeStruct((8, D), _f32),
    )(xuh, xih, xuc, xic)


def _main_body(t, xu_h, xu_c, xi_h, xi_c, m_uh, m_uc, m_ih, m_ic, degs,
               eu, ei, scv, shv,
               Wu_eh, Wu_ec, Wi_eh, Wi_ec, Wz_u, Wh_u, Wc_u, Wz_i, Wh_i, Wc_i,
               zu_enc, zi_enc, xu_t, xi_t):
    decay = jnp.exp(-t[0, 0])
    sc = scv[...]
    sh = shv[...]
    dg = degs[...]

    def bn(x, r):
        return x[...] * sc[r:r + 1, :] + sh[r:r + 1, :]

    def evo(xraw, rx, m, rg, p, W, e):
        h = (decay * bn(xraw, rx) + m[...] * sc[rg:rg + 1, :]
             + dg[:, p:p + 1] * sh[rg:rg + 1, :])
        return jnp.tanh(jnp.dot(h, W[...], preferred_element_type=_f32)) + e

    def side(xrh, xrc, rxh, rxc, mh, mc, rgh, rgc, ph, pc, e,
             Weh, Wec, Wz, Wh, Wc, zenc, xt):
        ev = e[...]
        xmh = evo(xrh, rxh, mh, rgh, ph, Weh, ev)
        xmc = evo(xrc, rxc, mc, rgc, pc, Wec, ev)

        def two(Wr):
            return (jnp.dot(xmh, Wr[pl.ds(0, D), :],
                            preferred_element_type=_f32) +
                    jnp.dot(xmc, Wr[pl.ds(D, D), :],
                            preferred_element_type=_f32))

        zenc[:, pl.ds(0, D)] = jnp.tanh(two(Wz))
        zenc[:, pl.ds(D, D)] = ev
        xt[:, pl.ds(0, D)] = jnp.tanh(two(Wh))
        xt[:, pl.ds(D, D)] = jnp.tanh(two(Wc))

    side(xu_h, xu_c, 0, 2, m_uh, m_uc, 1, 3, 0, 2, eu,
         Wu_eh, Wu_ec, Wz_u, Wh_u, Wc_u, zu_enc, xu_t)
    side(xi_h, xi_c, 1, 3, m_ih, m_ic, 0, 2, 1, 3, ei,
         Wi_eh, Wi_ec, Wz_i, Wh_i, Wc_i, zi_enc, xi_t)


def _tc_main(t2, xu_h, xu_c, xi_h, xi_c, m_uh, m_uc, m_ih, m_ic, degs,
             eu, ei, scales, shifts,
             Wu_eh, Wu_ec, Wi_eh, Wi_ec, Wz_u, Wh_u, Wc_u, Wz_i, Wh_i, Wc_i):
    spec = pl.BlockSpec((_RB, D), lambda i: (i, 0))
    spec2 = pl.BlockSpec((_RB, 2 * D), lambda i: (i, 0))
    dspec = pl.BlockSpec((_RB, 4), lambda i: (i, 0))
    w1 = pl.BlockSpec((D, D), lambda i: (0, 0))
    w2 = pl.BlockSpec((2 * D, D), lambda i: (0, 0))
    cspec = pl.BlockSpec((4, D), lambda i: (0, 0))
    tspec = pl.BlockSpec((1, 1), lambda i: (0, 0))
    return pl.pallas_call(
        _main_body,
        grid=(_NB,),
        in_specs=[tspec] + [spec] * 8 + [dspec] + [spec] * 2 + [cspec] * 2
                 + [w1] * 4 + [w2] * 6,
        out_specs=[spec2, spec2, spec2, spec2],
        out_shape=[
            jax.ShapeDtypeStruct((NU, 2 * D), _f32),
            jax.ShapeDtypeStruct((NI, 2 * D), _f32),
            jax.ShapeDtypeStruct((NU, 2 * D), _f32),
            jax.ShapeDtypeStruct((NI, 2 * D), _f32),
        ],
    )(t2, xu_h, xu_c, xi_h, xi_c, m_uh, m_uc, m_ih, m_ic, degs,
      eu, ei, scales, shifts,
      Wu_eh, Wu_ec, Wi_eh, Wi_ec, Wz_u, Wh_u, Wc_u, Wz_i, Wh_i, Wc_i)


def _loss_body(zup, zip_, zun, zin, Wpu, Wpi, o):
    pu = jnp.dot(zup[...], Wpu[...], preferred_element_type=_f32)
    pi = jnp.dot(zip_[...], Wpi[...], preferred_element_type=_f32)
    cols = [jnp.sum(pu * pi, 1, keepdims=True)]
    for k in range(K):
        pik = jnp.dot(zin[:, pl.ds(k * 2 * D, 2 * D)], Wpi[...],
                      preferred_element_type=_f32)
        cols.append(jnp.sum(pu * pik, 1, keepdims=True))
    for k in range(K):
        puk = jnp.dot(zun[:, pl.ds(k * 2 * D, 2 * D)], Wpu[...],
                      preferred_element_type=_f32)
        cols.append(jnp.sum(puk * pi, 1, keepdims=True))
    scores = jnp.concatenate(cols, axis=1)
    m = jnp.max(scores, 1, keepdims=True)
    lse = jnp.log(jnp.sum(jnp.exp(scores - m), 1, keepdims=True)) + m
    loss = jnp.mean(lse[:, 0] - scores[:, 0])
    o[...] = jnp.broadcast_to(loss, (8, 128))


def _tc_loss(zu_pos, zi_pos, zu_neg2, zi_neg2, Wp_u, Wp_i):
    full = lambda s: pl.BlockSpec(s, lambda: (0, 0))
    return pl.pallas_call(
        _loss_body,
        in_specs=[full((B, 2 * D)), full((B, 2 * D)),
                  full((B, 2 * D * K)), full((B, 2 * D * K)),
                  full((2 * D, D)), full((2 * D, D))],
        out_specs=full((8, 128)),
        out_shape=jax.ShapeDtypeStruct((8, 128), _f32),
    )(zu_pos, zi_pos, zu_neg2, zi_neg2, Wp_u, Wp_i)


def _final_body(xu_t, xi_t, mu_u, mu_i,
                Wu_uh, Wi_uh, Wu_uc, Wi_uc, ou_h, oi_h, ou_c, oi_c):
    xu = xu_t[...]
    xi = xi_t[...]
    mu = mu_u[...]
    mi = mu_i[...]
    for x, m, w, o in (
            (xu[:, :D], mu[:, :D], Wu_uh, ou_h),
            (xi[:, :D], mi[:, :D], Wi_uh, oi_h),
            (xu[:, D:], mu[:, D:], Wu_uc, ou_c),
            (xi[:, D:], mi[:, D:], Wi_uc, oi_c)):
        o[...] = x + jnp.tanh(
            jnp.dot(m, w[...], preferred_element_type=_f32))


def _tc_final(xu_t, xi_t, mu_u, mu_i, Wu_uh, Wi_uh, Wu_uc, Wi_uc):
    spec = pl.BlockSpec((_RB, D), lambda i: (i, 0))
    spec2 = pl.BlockSpec((_RB, 2 * D), lambda i: (i, 0))
    w1 = pl.BlockSpec((D, D), lambda i: (0, 0))
    return pl.pallas_call(
        _final_body,
        grid=(_NB,),
        in_specs=[spec2] * 4 + [w1] * 4,
        out_specs=[spec] * 4,
        out_shape=[jax.ShapeDtypeStruct((NU, D), _f32)] * 4,
    )(xu_t, xi_t, mu_u, mu_i, Wu_uh, Wi_uh, Wu_uc, Wi_uc)


def kernel(t_diff, adj_his, adj_ctx, adj_tgt_i2u, adj_tgt_u2i, tgt_u, tgt_i,
           tgt_u_neg, tgt_i_neg, xu_in_his, xi_in_his, xu_in_ctx, xi_in_ctx,
           embeds_u, embeds_i, g_uh, b_uh, g_ih, b_ih, g_uc, b_uc, g_ic, b_ic,
           Wu_eh, Wi_eh, Wu_ec, Wi_ec, Wu_uh, Wi_uh, Wu_uc, Wi_uc,
           Wz_u, Wh_u, Wc_u, Wz_i, Wh_i, Wc_i, Wp_u, Wp_i):
    stats = _tc_stats(xu_in_his, xi_in_his, xu_in_ctx, xi_in_ctx)
    sums = stats[0::2, :] / NU
    sqs = stats[1::2, :] / NU
    var = sqs - sums * sums
    g = jnp.stack([g_uh, g_ih, g_uc, g_ic])
    bb = jnp.stack([b_uh, b_ih, b_uc, b_ic])
    scales = g / jnp.sqrt(var + 1e-5)
    shifts = bb - sums * scales

    pad = E_PAD - E
    padsrc = (jnp.arange(pad, dtype=_i32) * 61) % NU
    padbad = NU + (jnp.arange(pad, dtype=_i32) % (ACC2 - NU))

    def prep(row):
        src = jnp.concatenate([row, padsrc]).reshape(-1, 128)
        dst = jnp.concatenate([row, padbad]).reshape(-1, 128)
        return src, dst

    s_h_s, s_h_d = prep(adj_his[0])
    d_h_s, d_h_d = prep(adj_his[1])
    s_c_s, s_c_d = prep(adj_ctx[0])
    d_c_s, d_c_d = prep(adj_ctx[1])
    zrows = jnp.zeros((STRIPE, D), _f32)
    zrows32 = jnp.zeros((WSTRIPE, HD), _f32)
    zdeg = jnp.zeros((WSTRIPE,), _f32)

    (m_uh, m_ih, m_uc, m_ic, dg_uh, dg_ih, dg_uc, dg_ic,
     _t0, _t1, _t2, _t3) = _sc_msgs(
        s_h_s, s_h_d, d_h_s, d_h_d, s_c_s, s_c_d, d_c_s, d_c_d,
        xi_in_his, xu_in_his, xi_in_ctx, xu_in_ctx, zrows32, zdeg)
    degs = jnp.stack([dg_uh, dg_ih, dg_uc, dg_ic], axis=1)

    t2 = t_diff.reshape(1, 1)
    zu_enc, zi_enc, xu_t, xi_t = _tc_main(
        t2, xu_in_his, xu_in_ctx, xi_in_his, xi_in_ctx,
        m_uh, m_uc, m_ih, m_ic, degs,
        embeds_u, embeds_i, scales, shifts,
        Wu_eh, Wu_ec, Wi_eh, Wi_ec, Wz_u, Wh_u, Wc_u, Wz_i, Wh_i, Wc_i)
    xu_th, xu_tc = xu_t[:, :D], xu_t[:, D:]
    xi_th, xi_tc = xi_t[:, :D], xi_t[:, D:]

    zu_pos, zi_pos, zu_neg, zi_neg, mu_u, mu_i = _sc_targets(
        zu_enc, zi_enc, xu_th, xi_th, xu_tc, xi_tc,
        tgt_u.reshape(-1, 128), tgt_i.reshape(-1, 128),
        tgt_u_neg.reshape(-1, 128), tgt_i_neg.reshape(-1, 128),
        adj_tgt_i2u[0].reshape(-1, 128), adj_tgt_i2u[1].reshape(-1, 128),
        adj_tgt_u2i[0].reshape(-1, 128), adj_tgt_u2i[1].reshape(-1, 128),
        zrows)

    lossmat = _tc_loss(zu_pos, zi_pos,
                       zu_neg.reshape(B, 2 * D * K),
                       zi_neg.reshape(B, 2 * D * K), Wp_u, Wp_i)
    loss = lossmat[0, 0]

    ou_h, oi_h, ou_c, oi_c = _tc_final(
        xu_t, xi_t, mu_u, mu_i, Wu_uh, Wi_uh, Wu_uc, Wi_uc)

    return (loss, zu_pos, zi_enc, ou_h, oi_h, ou_c, oi_c)
